# Initial kernel scaffold; baseline (speedup 1.0000x reference)
import jax
import jax.numpy as jnp
from jax.experimental import pallas as pl


def kernel(boxes, scores, gt_bbox):
    raise NotImplementedError("write your pallas kernel here")


# speedup vs baseline: 4.6206x; 4.6206x over previous
import jax
import jax.numpy as jnp
from jax import lax
from jax.experimental import pallas as pl
from jax.experimental.pallas import tpu as pltpu

_N = 20000
_G = 20
_PRE = 2000
_POST = 1000
_T = 0.7
_NB = 20
_PADN = _NB * 1024
_NEG = -3.0e38


def _fused_body(gt_ref, bx0_ref, by0_ref, bx1_ref, by1_ref, sc_ref,
                out_ref,
                ss_ref, t0_ref, t1_ref, t2_ref, t3_ref,
                k0_ref, k1_ref, k2_ref, k3_ref, bm_ref, kcnt_ref):
    f32 = jnp.float32
    fl8 = (lax.broadcasted_iota(jnp.int32, (8, 128), 0) * 128
           + lax.broadcasted_iota(jnp.int32, (8, 128), 1))
    lane1 = lax.broadcasted_iota(jnp.int32, (1, 128), 1)

    out_ref[...] = jnp.zeros((9, 8, 128), f32)
    far = jnp.full((8, 128), 1e7, f32)
    k0_ref[...] = far
    k1_ref[...] = far
    k2_ref[...] = far + 1.0
    k3_ref[...] = far + 1.0
    bm_ref[...] = jnp.full((1, 128), _NEG, f32)
    kcnt_ref[0] = 0

    def p1(v, carry):
        x0 = bx0_ref[v]
        y0 = by0_ref[v]
        x1 = bx1_ref[v]
        y1 = by1_ref[v]
        area = (x1 - x0) * (y1 - y0)
        best = jnp.full((8, 128), -1.0, f32)
        mg0 = jnp.zeros((8, 128), f32)
        mg1 = jnp.zeros((8, 128), f32)
        mg2 = jnp.ones((8, 128), f32)
        mg3 = jnp.ones((8, 128), f32)
        for g in range(_G):
            g0 = gt_ref[g, 0]
            g1 = gt_ref[g, 1]
            g2 = gt_ref[g, 2]
            g3 = gt_ref[g, 3]
            ga = (g2 - g0) * (g3 - g1)
            iw = jnp.maximum(jnp.minimum(x1, g2) - jnp.maximum(x0, g0), 0.0)
            ih = jnp.maximum(jnp.minimum(y1, g3) - jnp.maximum(y0, g1), 0.0)
            inter = iw * ih
            iou = inter / (area + ga - inter)
            upd = iou > best
            best = jnp.where(upd, iou, best)
            mg0 = jnp.where(upd, g0, mg0)
            mg1 = jnp.where(upd, g1, mg1)
            mg2 = jnp.where(upd, g2, mg2)
            mg3 = jnp.where(upd, g3, mg3)
        pw = x1 - x0
        ph = y1 - y0
        px = x0 + 0.5 * pw
        py = y0 + 0.5 * ph
        gw = mg2 - mg0
        gh = mg3 - mg1
        gx = mg0 + 0.5 * gw
        gy = mg1 + 0.5 * gh
        t0_ref[v] = (gx - px) / pw
        t1_ref[v] = (gy - py) / ph
        t2_ref[v] = jnp.log(gw / pw)
        t3_ref[v] = jnp.log(gh / ph)
        s = sc_ref[v]
        ss_ref[v] = s
        bm_ref[...] = jnp.where(lane1 == v, jnp.max(s), bm_ref[...])
        return carry

    lax.fori_loop(0, _NB, p1, 0)

    def sel(it, carry):
        K = kcnt_ref[0]

        @pl.when(K < _POST)
        def _():
            bm = bm_ref[...]
            m = jnp.max(bm)
            v = jnp.min(jnp.where(bm == m, lane1, 1 << 30))
            blk = ss_ref[v]
            w = jnp.min(jnp.where(blk == m, fl8, 1 << 30))
            sel_mask = fl8 == w

            def pick(ref):
                return jnp.max(jnp.where(sel_mask, ref[v], _NEG))

            x0 = pick(bx0_ref)
            y0 = pick(by0_ref)
            x1 = pick(bx1_ref)
            y1 = pick(by1_ref)
            a = (x1 - x0) * (y1 - y0)
            K0 = k0_ref[...]
            K1 = k1_ref[...]
            K2 = k2_ref[...]
            K3 = k3_ref[...]
            iw = jnp.maximum(jnp.minimum(K2, x1) - jnp.maximum(K0, x0), 0.0)
            ih = jnp.maximum(jnp.minimum(K3, y1) - jnp.maximum(K1, y0), 0.0)
            inter = iw * ih
            ka = (K2 - K0) * (K3 - K1)
            ov = inter - _T * (ka + a - inter)
            sup = jnp.max(ov) > 0.0

            @pl.when(jnp.logical_not(sup))
            def _():
                wm = fl8 == K
                k0_ref[...] = jnp.where(wm, x0, K0)
                k1_ref[...] = jnp.where(wm, y0, K1)
                k2_ref[...] = jnp.where(wm, x1, K2)
                k3_ref[...] = jnp.where(wm, y1, K3)
                out_ref[0] = jnp.where(wm, x0, out_ref[0])
                out_ref[1] = jnp.where(wm, y0, out_ref[1])
                out_ref[2] = jnp.where(wm, x1, out_ref[2])
                out_ref[3] = jnp.where(wm, y1, out_ref[3])
                out_ref[4] = jnp.where(wm, pick(t0_ref), out_ref[4])
                out_ref[5] = jnp.where(wm, pick(t1_ref), out_ref[5])
                out_ref[6] = jnp.where(wm, pick(t2_ref), out_ref[6])
                out_ref[7] = jnp.where(wm, pick(t3_ref), out_ref[7])
                out_ref[8] = jnp.where(wm, m, out_ref[8])
                kcnt_ref[0] = K + 1

            nb = jnp.where(sel_mask, _NEG, blk)
            ss_ref[v] = nb
            bm_ref[...] = jnp.where(lane1 == v, jnp.max(nb), bm)

        return carry

    lax.fori_loop(0, _PRE, sel, 0)


def kernel(boxes, scores, gt_bbox):
    pad_boxes = jnp.broadcast_to(jnp.array([0.0, 0.0, 1.0, 1.0], jnp.float32),
                                 (_PADN - _N, 4))
    b = jnp.concatenate([boxes, pad_boxes], axis=0)
    cols = b.T.reshape(4, _NB, 8, 128)
    sp = jnp.concatenate(
        [scores, jnp.full((_PADN - _N,), _NEG, jnp.float32)]).reshape(_NB, 8, 128)

    out = pl.pallas_call(
        _fused_body,
        out_shape=jax.ShapeDtypeStruct((9, 8, 128), jnp.float32),
        in_specs=[
            pl.BlockSpec(memory_space=pltpu.SMEM),
            pl.BlockSpec(memory_space=pltpu.VMEM),
            pl.BlockSpec(memory_space=pltpu.VMEM),
            pl.BlockSpec(memory_space=pltpu.VMEM),
            pl.BlockSpec(memory_space=pltpu.VMEM),
            pl.BlockSpec(memory_space=pltpu.VMEM),
        ],
        out_specs=pl.BlockSpec(memory_space=pltpu.VMEM),
        scratch_shapes=[
            pltpu.VMEM((_NB, 8, 128), jnp.float32),
            pltpu.VMEM((_NB, 8, 128), jnp.float32),
            pltpu.VMEM((_NB, 8, 128), jnp.float32),
            pltpu.VMEM((_NB, 8, 128), jnp.float32),
            pltpu.VMEM((_NB, 8, 128), jnp.float32),
            pltpu.VMEM((8, 128), jnp.float32),
            pltpu.VMEM((8, 128), jnp.float32),
            pltpu.VMEM((8, 128), jnp.float32),
            pltpu.VMEM((8, 128), jnp.float32),
            pltpu.VMEM((1, 128), jnp.float32),
            pltpu.SMEM((1,), jnp.int32),
        ],
    )(gt_bbox, cols[0], cols[1], cols[2], cols[3], sp)

    return out.reshape(9, _NB * 1024 // _NB)[:, :_POST].T


# speedup vs baseline: 45.5627x; 9.8608x over previous
import jax
import jax.numpy as jnp
from jax import lax
from jax.experimental import pallas as pl
from jax.experimental.pallas import tpu as pltpu

_N = 20000
_G = 20
_PRE = 2000
_POST = 1000
_T = 0.7
_NB = 20
_PADN = _NB * 1024
_CAP = 4096
_NEGI = jnp.float32(-jnp.inf)


def _k1_body(sc_ref, out_ref):
    i32 = jnp.int32
    b = lax.bitcast_convert_type(sc_ref[...], i32)
    key = jnp.where(b < 0, (-2147483648) - b, b)
    cnt_nonneg = jnp.sum((key >= 0).astype(i32))
    neg = cnt_nonneg < _PRE
    lo0 = jnp.where(neg, -2139095040, 0)
    hi0 = jnp.where(neg, -1, 2139095039)

    def bs(i, lohi):
        lo, hi = lohi
        mid = lo + (hi - lo + 1) // 2
        cnt = jnp.sum((key >= mid).astype(i32))
        ok = cnt >= _PRE
        return (jnp.where(ok, mid, lo), jnp.where(ok, hi, mid - 1))

    tau, _ = lax.fori_loop(0, 31, bs, (lo0, hi0))
    cnt_gt = jnp.sum((key > tau).astype(i32))
    need = _PRE - cnt_gt
    r = lax.broadcasted_iota(i32, (8, 128), 0)
    out_ref[...] = jnp.where(r == 0, tau, jnp.where(r == 1, need, 0))


def _k1_call(sp):
    return pl.pallas_call(
        _k1_body,
        out_shape=jax.ShapeDtypeStruct((8, 128), jnp.int32),
        in_specs=[pl.BlockSpec(memory_space=pltpu.VMEM)],
        out_specs=pl.BlockSpec(memory_space=pltpu.VMEM),
    )(sp)


def _k2_standin(spf, bx0, by0, bx1, by1, tau, needn):
    i32 = jnp.int32
    b = lax.bitcast_convert_type(spf, i32)
    key = jnp.where(b < 0, (-2147483648) - b, b)
    gt = key > tau
    eq = key == tau
    eqrank = jnp.cumsum(eq.astype(i32)) - 1
    emit = gt | (eq & (eqrank < needn))
    order = jnp.argsort(jnp.where(emit, jnp.arange(_PADN), 1 << 30))
    sel = order[:_CAP]
    valid = emit[sel]
    sc_c = jnp.where(valid, spf[sel], _NEGI)
    x0_c = jnp.where(valid, bx0[sel], 1.0e7)
    y0_c = jnp.where(valid, by0[sel], 1.0e7)
    x1_c = jnp.where(valid, bx1[sel], 1.0e7 + 1.0)
    y1_c = jnp.where(valid, by1[sel], 1.0e7 + 1.0)
    ix_c = jnp.where(valid, sel.astype(i32), (1 << 22))
    return sc_c, ix_c, x0_c, y0_c, x1_c, y1_c


def _k3_body(gt_ref, sc_ref, ix_ref, bx0_ref, by0_ref, bx1_ref, by1_ref,
             out_ref, c_ref, pt_ref):
    f32 = jnp.float32
    bf16 = jnp.bfloat16
    i32 = jnp.int32
    S, L = 32, 128
    s_io = lax.broadcasted_iota(i32, (S, L), 0)
    l_io = lax.broadcasted_iota(i32, (S, L), 1)
    f_io = s_io * L + l_io

    def xshuf(a, d):
        if d < L:
            fwd = pltpu.roll(a, L - d, 1)
            bwd = pltpu.roll(a, d, 1)
            bit = (l_io & d) == 0
        else:
            r = d // L
            fwd = pltpu.roll(a, S - r, 0)
            bwd = pltpu.roll(a, r, 0)
            bit = (s_io & r) == 0
        return jnp.where(bit, fwd, bwd)

    arrs = [sc_ref[...], ix_ref[...], bx0_ref[...], by0_ref[...],
            bx1_ref[...], by1_ref[...]]
    for kk in range(1, 13):
        size = 1 << kk
        for j in range(kk - 1, -1, -1):
            d = 1 << j
            p = [xshuf(a, d) for a in arrs]
            plt = (p[0] > arrs[0]) | ((p[0] == arrs[0]) & (p[1] < arrs[1]))
            lower = (f_io & d) == 0
            up = (f_io & size) == 0
            take = (lower == up) == plt
            arrs = [jnp.where(take, pa, a) for pa, a in zip(p, arrs)]
    sc = arrs[0][:16]
    x0 = arrs[2][:16]
    y0 = arrs[3][:16]
    x1 = arrs[4][:16]
    y1 = arrs[5][:16]

    area = (x1 - x0) * (y1 - y0)
    best = jnp.full((16, L), -1.0, f32)
    mg0 = jnp.zeros((16, L), f32)
    mg1 = jnp.zeros((16, L), f32)
    mg2 = jnp.ones((16, L), f32)
    mg3 = jnp.ones((16, L), f32)
    for g in range(_G):
        g0 = gt_ref[g, 0]
        g1 = gt_ref[g, 1]
        g2 = gt_ref[g, 2]
        g3 = gt_ref[g, 3]
        ga = (g2 - g0) * (g3 - g1)
        iw = jnp.maximum(jnp.minimum(x1, g2) - jnp.maximum(x0, g0), 0.0)
        ih = jnp.maximum(jnp.minimum(y1, g3) - jnp.maximum(y0, g1), 0.0)
        inter = iw * ih
        iou = inter / (area + ga - inter)
        upd = iou > best
        best = jnp.where(upd, iou, best)
        mg0 = jnp.where(upd, g0, mg0)
        mg1 = jnp.where(upd, g1, mg1)
        mg2 = jnp.where(upd, g2, mg2)
        mg3 = jnp.where(upd, g3, mg3)
    pw = x1 - x0
    ph = y1 - y0
    px = x0 + 0.5 * pw
    py = y0 + 0.5 * ph
    gw = mg2 - mg0
    gh = mg3 - mg1
    gx = mg0 + 0.5 * gw
    gy = mg1 + 0.5 * gh
    tx = (gx - px) / pw
    ty = (gy - py) / ph
    tw = jnp.log(gw / pw)
    th = jnp.log(gh / ph)

    c_ref[...] = jnp.zeros((2048, 2048), bf16)
    li_t = lax.broadcasted_iota(i32, (L, L), 0)
    lj_t = lax.broadcasted_iota(i32, (L, L), 1)
    for si in range(16):
        rx0 = x0[si:si + 1, :].T
        ry0 = y0[si:si + 1, :].T
        rx1 = x1[si:si + 1, :].T
        ry1 = y1[si:si + 1, :].T
        ra = area[si:si + 1, :].T
        for sj in range(si, 16):
            cx0 = x0[sj:sj + 1, :]
            cy0 = y0[sj:sj + 1, :]
            cx1 = x1[sj:sj + 1, :]
            cy1 = y1[sj:sj + 1, :]
            ca = area[sj:sj + 1, :]
            iw = jnp.maximum(jnp.minimum(rx1, cx1)
                             - jnp.maximum(rx0, cx0), 0.0)
            ih = jnp.maximum(jnp.minimum(ry1, cy1)
                             - jnp.maximum(ry0, cy0), 0.0)
            inter = iw * ih
            conf = inter > _T * (ra + ca - inter)
            m = conf & ((sj * L + lj_t) > (si * L + li_t))
            c_ref[si * L:(si + 1) * L, sj * L:(sj + 1) * L] = m.astype(bf16)

    def cond(c):
        alive, kept = c
        return jnp.sum(alive) > 0.0

    def rbody(c):
        alive, kept = c
        al8 = jnp.broadcast_to(alive, (8, 2048)).astype(bf16)
        ear = jnp.dot(al8, c_ref[...], preferred_element_type=f32)[0:1]
        frontier = (alive > 0.0) & (ear <= 0.5)
        fr8 = jnp.broadcast_to(frontier.astype(f32), (8, 2048)).astype(bf16)
        spread = jnp.dot(fr8, c_ref[...], preferred_element_type=f32)[0:1] > 0.5
        kept = jnp.where(frontier, 1.0, kept)
        alive = jnp.where(frontier | spread, 0.0, alive)
        return alive, kept

    alive0 = jnp.ones((1, 2048), f32)
    kept0 = jnp.zeros((1, 2048), f32)
    _, kept = lax.while_loop(cond, rbody, (alive0, kept0))

    l2 = lax.broadcasted_iota(i32, (1, 2048), 1)
    keptm = (kept > 0.0) & (l2 < _PRE)
    kv = keptm.astype(f32)
    run = kv
    for dd in (1, 2, 4, 8, 16, 32, 64, 128, 256, 512, 1024):
        run = run + jnp.where(l2 >= dd, pltpu.roll(run, dd, 1), 0.0)
    pos = run - kv

    o_io = lax.broadcasted_iota(i32, (L, 1024), 1).astype(f32)
    for si in range(16):
        ps = pos[0:1, si * L:(si + 1) * L].T
        ks = kv[0:1, si * L:(si + 1) * L].T
        pt_ref[si * L:(si + 1) * L, :] = \
            ((ps == o_io) & (ks > 0.5) & (ps < float(_POST))).astype(bf16)

    r16 = lax.broadcasted_iota(i32, (16, L), 0) * L \
        + lax.broadcasted_iota(i32, (16, L), 1)
    sc_s = jnp.where(r16 < _PRE, sc, 0.0)

    acc = jnp.zeros((16, 1024), f32)
    for si in range(16):
        dts = jnp.concatenate(
            [x0[si:si + 1], y0[si:si + 1], x1[si:si + 1], y1[si:si + 1],
             tx[si:si + 1], ty[si:si + 1], tw[si:si + 1], th[si:si + 1],
             sc_s[si:si + 1], jnp.zeros((7, L), f32)], axis=0)
        acc = acc + jnp.dot(dts, pt_ref[si * L:(si + 1) * L, :].astype(f32),
                            precision=lax.Precision.HIGHEST,
                            preferred_element_type=f32)
    out_ref[...] = acc


def _k3_call(gt_bbox, sc_c, ix_c, x0_c, y0_c, x1_c, y1_c):
    return pl.pallas_call(
        _k3_body,
        out_shape=jax.ShapeDtypeStruct((16, 1024), jnp.float32),
        in_specs=[
            pl.BlockSpec(memory_space=pltpu.SMEM),
            pl.BlockSpec(memory_space=pltpu.VMEM),
            pl.BlockSpec(memory_space=pltpu.VMEM),
            pl.BlockSpec(memory_space=pltpu.VMEM),
            pl.BlockSpec(memory_space=pltpu.VMEM),
            pl.BlockSpec(memory_space=pltpu.VMEM),
            pl.BlockSpec(memory_space=pltpu.VMEM),
        ],
        out_specs=pl.BlockSpec(memory_space=pltpu.VMEM),
        scratch_shapes=[
            pltpu.VMEM((2048, 2048), jnp.bfloat16),
            pltpu.VMEM((2048, 1024), jnp.bfloat16),
        ],
    )(gt_bbox, sc_c.reshape(32, 128), ix_c.reshape(32, 128),
      x0_c.reshape(32, 128), y0_c.reshape(32, 128),
      x1_c.reshape(32, 128), y1_c.reshape(32, 128))


def kernel(boxes, scores, gt_bbox):
    spf = jnp.concatenate(
        [scores, jnp.full((_PADN - _N,), _NEGI, jnp.float32)])
    sp = spf.reshape(_NB, 8, 128)
    k1 = _k1_call(sp)
    tau = k1[0, 0]
    needn = k1[1, 0]
    bx0 = jnp.concatenate([boxes[:, 0], jnp.zeros((_PADN - _N,), jnp.float32)])
    by0 = jnp.concatenate([boxes[:, 1], jnp.zeros((_PADN - _N,), jnp.float32)])
    bx1 = jnp.concatenate([boxes[:, 2], jnp.ones((_PADN - _N,), jnp.float32)])
    by1 = jnp.concatenate([boxes[:, 3], jnp.ones((_PADN - _N,), jnp.float32)])
    sc_c, ix_c, x0_c, y0_c, x1_c, y1_c = _k2_standin(
        spf, bx0, by0, bx1, by1, tau, needn)
    out = _k3_call(gt_bbox, sc_c, ix_c, x0_c, y0_c, x1_c, y1_c)
    return out[:9, :_POST].T


# speedup vs baseline: 67.5823x; 1.4833x over previous
import functools

import jax
import jax.numpy as jnp
from jax import lax
from jax.experimental import pallas as pl
from jax.experimental.pallas import tpu as pltpu
from jax.experimental.pallas import tpu_sc as plsc

_N = 20000
_G = 20
_PRE = 2000
_POST = 1000
_T = 0.7
_NB = 20
_PADN = _NB * 1024
_CAP = 4096
_NEGI = jnp.float32(-jnp.inf)


def _k1_body(sc_ref, out_ref):
    i32 = jnp.int32
    b = lax.bitcast_convert_type(sc_ref[...], i32)
    key = jnp.where(b < 0, (-2147483648) - b, b)
    cnt_nonneg = jnp.sum((key >= 0).astype(i32))
    neg = cnt_nonneg < _PRE
    lo0 = jnp.where(neg, -2139095040, 0)
    hi0 = jnp.where(neg, -1, 2139095039)

    def bs(i, lohi):
        lo, hi = lohi
        mid = lo + (hi - lo + 1) // 2
        cnt = jnp.sum((key >= mid).astype(i32))
        ok = cnt >= _PRE
        return (jnp.where(ok, mid, lo), jnp.where(ok, hi, mid - 1))

    tau, _ = lax.fori_loop(0, 31, bs, (lo0, hi0))
    cnt_gt = jnp.sum((key > tau).astype(i32))
    need = _PRE - cnt_gt
    taubits = jnp.where(tau >= 0, tau, (-2147483648) - tau)
    r = lax.broadcasted_iota(i32, (8, 128), 0)
    out_ref[...] = jnp.where(r == 0, taubits, jnp.where(r == 1, need, 0))


def _k1_call(sp):
    return pl.pallas_call(
        _k1_body,
        out_shape=jax.ShapeDtypeStruct((8, 128), jnp.int32),
        in_specs=[pl.BlockSpec(memory_space=pltpu.VMEM)],
        out_specs=pl.BlockSpec(memory_space=pltpu.VMEM),
    )(sp)


_STRIPE = 1280
_NTILES = 16
_RUP = 128


def _k2_body(sc_hbm, x0_hbm, y0_hbm, x1_hbm, y1_hbm, tau_hbm, need_hbm,
             osc, ox0, oy0, ox1, oy1, oix,
             sbuf, x0b, y0b, x1b, y1b,
             lsc, lx0, ly0, lx1, ly1, lix,
             taub, needb, cntb, allcnt, shared):
    i32 = jnp.int32
    f32 = jnp.float32
    c = lax.axis_index("c")
    w = lax.axis_index("s")

    @pl.when(c == 0)
    def _():
        base = pl.multiple_of(w * _STRIPE, 256)
        pltpu.sync_copy(sc_hbm.at[pl.ds(base, _STRIPE)], sbuf)
        pltpu.sync_copy(x0_hbm.at[pl.ds(base, _STRIPE)], x0b)
        pltpu.sync_copy(y0_hbm.at[pl.ds(base, _STRIPE)], y0b)
        pltpu.sync_copy(x1_hbm.at[pl.ds(base, _STRIPE)], x1b)
        pltpu.sync_copy(y1_hbm.at[pl.ds(base, _STRIPE)], y1b)
        pltpu.sync_copy(tau_hbm, taub)
        pltpu.sync_copy(need_hbm, needb)
        tauv = taub[...]
        needv = needb[...]
        lane = lax.broadcasted_iota(i32, (16,), 0)

        def pf(k, carry):
            lsc[pl.ds(k * 16, 16)] = jnp.full((16,), -jnp.inf, f32)
            lx0[pl.ds(k * 16, 16)] = jnp.full((16,), 1.0e7, f32)
            ly0[pl.ds(k * 16, 16)] = jnp.full((16,), 1.0e7, f32)
            lx1[pl.ds(k * 16, 16)] = jnp.full((16,), 1.0e7 + 1.0, f32)
            ly1[pl.ds(k * 16, 16)] = jnp.full((16,), 1.0e7 + 1.0, f32)
            lix[pl.ds(k * 16, 16)] = jnp.full((16,), 1 << 22, i32)
            return carry

        lax.fori_loop(0, 82, pf, 0)

        tb = pl.multiple_of(w * 256, 256)
        pltpu.sync_copy(lsc.at[pl.ds(0, 256)], osc.at[pl.ds(tb, 256)])
        pltpu.sync_copy(lx0.at[pl.ds(0, 256)], ox0.at[pl.ds(tb, 256)])
        pltpu.sync_copy(ly0.at[pl.ds(0, 256)], oy0.at[pl.ds(tb, 256)])
        pltpu.sync_copy(lx1.at[pl.ds(0, 256)], ox1.at[pl.ds(tb, 256)])
        pltpu.sync_copy(ly1.at[pl.ds(0, 256)], oy1.at[pl.ds(tb, 256)])
        pltpu.sync_copy(lix.at[pl.ds(0, 256)], oix.at[pl.ds(tb, 256)])

        def key_of(k):
            return sbuf[pl.ds(k * 16, 16)]

        def cnt(k, carry):
            cg, ce = carry
            key = key_of(k)
            cg = cg + plsc.all_reduce_population_count(key > tauv)
            ce = ce + plsc.all_reduce_population_count(key == tauv)
            return cg, ce

        z = jnp.zeros((16,), i32)
        cgv, cev = lax.fori_loop(0, 80, cnt, (z, z))
        cntb[...] = jnp.where(lane == 0, cgv, jnp.where(lane == 1, cev, 0))
        pltpu.sync_copy(cntb, shared.at[w])
        plsc.subcore_barrier()
        pltpu.sync_copy(shared, allcnt)

        needn = jnp.max(needv)

        def pfx(v, carry):
            eqp, off, myeqp, myoff = carry
            row = allcnt[v]
            cg = jnp.max(jnp.where(lane == 0, row, 0))
            ce = jnp.max(jnp.where(lane == 1, row, 0))
            n_v = cg + jnp.clip(needn - eqp, 0, ce)
            m_v = ((n_v + _RUP - 1) // _RUP) * _RUP
            myeqp = jnp.where(v == w, eqp, myeqp)
            myoff = jnp.where(v == w, off, myoff)
            return eqp + ce, off + m_v, myeqp, myoff

        zero = jnp.zeros((), i32)
        _, _, myeqp, myoff = lax.fori_loop(
            0, _NTILES, pfx, (zero, zero, zero, zero))

        def emit(k, carry):
            off, eqc = carry
            key = key_of(k)
            gtm = key > tauv
            eqm = key == tauv
            eqi = eqm.astype(i32)
            excl = plsc.cumsum(eqi) - eqi
            rank = myeqp + eqc + excl
            em = gtm | (eqm & (rank < needn))
            gidx = base + k * 16 + lane
            plsc.store_compressed(lsc.at[pl.ds(off, 16)],
                                  sbuf[pl.ds(k * 16, 16)], mask=em)
            plsc.store_compressed(lx0.at[pl.ds(off, 16)],
                                  x0b[pl.ds(k * 16, 16)], mask=em)
            plsc.store_compressed(ly0.at[pl.ds(off, 16)],
                                  y0b[pl.ds(k * 16, 16)], mask=em)
            plsc.store_compressed(lx1.at[pl.ds(off, 16)],
                                  x1b[pl.ds(k * 16, 16)], mask=em)
            plsc.store_compressed(ly1.at[pl.ds(off, 16)],
                                  y1b[pl.ds(k * 16, 16)], mask=em)
            plsc.store_compressed(lix.at[pl.ds(off, 16)], gidx, mask=em)
            npop = jnp.max(plsc.all_reduce_population_count(em))
            neq = jnp.max(plsc.all_reduce_population_count(eqm))
            return off + npop, eqc + neq

        n_w, _ = lax.fori_loop(0, 80, emit, (zero, zero))
        m_w = ((n_w + _RUP - 1) // _RUP) * _RUP

        def wr(g, carry):
            s = g * _RUP
            d = pl.multiple_of(myoff + g * _RUP, _RUP)
            pltpu.sync_copy(lsc.at[pl.ds(s, _RUP)], osc.at[pl.ds(d, _RUP)])
            pltpu.sync_copy(lx0.at[pl.ds(s, _RUP)], ox0.at[pl.ds(d, _RUP)])
            pltpu.sync_copy(ly0.at[pl.ds(s, _RUP)], oy0.at[pl.ds(d, _RUP)])
            pltpu.sync_copy(lx1.at[pl.ds(s, _RUP)], ox1.at[pl.ds(d, _RUP)])
            pltpu.sync_copy(ly1.at[pl.ds(s, _RUP)], oy1.at[pl.ds(d, _RUP)])
            pltpu.sync_copy(lix.at[pl.ds(s, _RUP)], oix.at[pl.ds(d, _RUP)])
            return carry

        lax.fori_loop(0, m_w // _RUP, wr, 0)


def _k2_call(spf, bx0, by0, bx1, by1, tau, needn):
    i32 = jnp.int32
    f32 = jnp.float32
    tauv = jnp.full((16,), lax.bitcast_convert_type(tau, f32), f32)
    needv = jnp.full((16,), needn, i32)
    fn = functools.partial(
        pl.kernel,
        mesh=plsc.VectorSubcoreMesh(core_axis_name="c", subcore_axis_name="s"),
        compiler_params=pltpu.CompilerParams(needs_layout_passes=False),
        out_type=[
            jax.ShapeDtypeStruct((_CAP,), f32),
            jax.ShapeDtypeStruct((_CAP,), f32),
            jax.ShapeDtypeStruct((_CAP,), f32),
            jax.ShapeDtypeStruct((_CAP,), f32),
            jax.ShapeDtypeStruct((_CAP,), f32),
            jax.ShapeDtypeStruct((_CAP,), i32),
        ],
        scratch_types=[
            pltpu.VMEM((_STRIPE,), f32),
            pltpu.VMEM((_STRIPE,), f32),
            pltpu.VMEM((_STRIPE,), f32),
            pltpu.VMEM((_STRIPE,), f32),
            pltpu.VMEM((_STRIPE,), f32),
            pltpu.VMEM((1312,), f32),
            pltpu.VMEM((1312,), f32),
            pltpu.VMEM((1312,), f32),
            pltpu.VMEM((1312,), f32),
            pltpu.VMEM((1312,), f32),
            pltpu.VMEM((1312,), i32),
            pltpu.VMEM((16,), f32),
            pltpu.VMEM((16,), i32),
            pltpu.VMEM((16,), i32),
            pltpu.VMEM((16, 16), i32),
            pltpu.VMEM_SHARED((16, 16), i32),
        ],
    )(_k2_body)
    osc, ox0, oy0, ox1, oy1, oix = fn(spf, bx0, by0, bx1, by1, tauv, needv)
    return osc, oix, ox0, oy0, ox1, oy1


def _k2_standin(spf, bx0, by0, bx1, by1, tau, needn):
    i32 = jnp.int32
    b = lax.bitcast_convert_type(spf, i32)
    key = jnp.where(b < 0, (-2147483648) - b, b)
    gt = key > tau
    eq = key == tau
    eqrank = jnp.cumsum(eq.astype(i32)) - 1
    emit = gt | (eq & (eqrank < needn))
    order = jnp.argsort(jnp.where(emit, jnp.arange(_PADN), 1 << 30))
    sel = order[:_CAP]
    valid = emit[sel]
    sc_c = jnp.where(valid, spf[sel], _NEGI)
    x0_c = jnp.where(valid, bx0[sel], 1.0e7)
    y0_c = jnp.where(valid, by0[sel], 1.0e7)
    x1_c = jnp.where(valid, bx1[sel], 1.0e7 + 1.0)
    y1_c = jnp.where(valid, by1[sel], 1.0e7 + 1.0)
    ix_c = jnp.where(valid, sel.astype(i32), (1 << 22))
    return sc_c, ix_c, x0_c, y0_c, x1_c, y1_c


def _k3_body(gt_ref, sc_ref, ix_ref, bx0_ref, by0_ref, bx1_ref, by1_ref,
             out_ref, c_ref, pt_ref):
    f32 = jnp.float32
    bf16 = jnp.bfloat16
    i32 = jnp.int32
    S, L = 32, 128
    s_io = lax.broadcasted_iota(i32, (S, L), 0)
    l_io = lax.broadcasted_iota(i32, (S, L), 1)
    f_io = s_io * L + l_io

    def xshuf(a, d):
        if d < L:
            fwd = pltpu.roll(a, L - d, 1)
            bwd = pltpu.roll(a, d, 1)
            bit = (l_io & d) == 0
        else:
            r = d // L
            fwd = pltpu.roll(a, S - r, 0)
            bwd = pltpu.roll(a, r, 0)
            bit = (s_io & r) == 0
        return jnp.where(bit, fwd, bwd)

    arrs = [sc_ref[...], ix_ref[...], bx0_ref[...], by0_ref[...],
            bx1_ref[...], by1_ref[...]]
    for kk in range(1, 13):
        size = 1 << kk
        for j in range(kk - 1, -1, -1):
            d = 1 << j
            p = [xshuf(a, d) for a in arrs]
            plt = (p[0] > arrs[0]) | ((p[0] == arrs[0]) & (p[1] < arrs[1]))
            lower = (f_io & d) == 0
            up = (f_io & size) == 0
            take = (lower == up) == plt
            arrs = [jnp.where(take, pa, a) for pa, a in zip(p, arrs)]
    sc = arrs[0][:16]
    x0 = arrs[2][:16]
    y0 = arrs[3][:16]
    x1 = arrs[4][:16]
    y1 = arrs[5][:16]

    area = (x1 - x0) * (y1 - y0)
    best = jnp.full((16, L), -1.0, f32)
    mg0 = jnp.zeros((16, L), f32)
    mg1 = jnp.zeros((16, L), f32)
    mg2 = jnp.ones((16, L), f32)
    mg3 = jnp.ones((16, L), f32)
    for g in range(_G):
        g0 = gt_ref[g, 0]
        g1 = gt_ref[g, 1]
        g2 = gt_ref[g, 2]
        g3 = gt_ref[g, 3]
        ga = (g2 - g0) * (g3 - g1)
        iw = jnp.maximum(jnp.minimum(x1, g2) - jnp.maximum(x0, g0), 0.0)
        ih = jnp.maximum(jnp.minimum(y1, g3) - jnp.maximum(y0, g1), 0.0)
        inter = iw * ih
        iou = inter / (area + ga - inter)
        upd = iou > best
        best = jnp.where(upd, iou, best)
        mg0 = jnp.where(upd, g0, mg0)
        mg1 = jnp.where(upd, g1, mg1)
        mg2 = jnp.where(upd, g2, mg2)
        mg3 = jnp.where(upd, g3, mg3)
    pw = x1 - x0
    ph = y1 - y0
    px = x0 + 0.5 * pw
    py = y0 + 0.5 * ph
    gw = mg2 - mg0
    gh = mg3 - mg1
    gx = mg0 + 0.5 * gw
    gy = mg1 + 0.5 * gh
    tx = (gx - px) / pw
    ty = (gy - py) / ph
    tw = jnp.log(gw / pw)
    th = jnp.log(gh / ph)

    c_ref[...] = jnp.zeros((2048, 2048), bf16)
    li_t = lax.broadcasted_iota(i32, (L, L), 0)
    lj_t = lax.broadcasted_iota(i32, (L, L), 1)
    for si in range(16):
        rx0 = x0[si:si + 1, :].T
        ry0 = y0[si:si + 1, :].T
        rx1 = x1[si:si + 1, :].T
        ry1 = y1[si:si + 1, :].T
        ra = area[si:si + 1, :].T
        for sj in range(si, 16):
            cx0 = x0[sj:sj + 1, :]
            cy0 = y0[sj:sj + 1, :]
            cx1 = x1[sj:sj + 1, :]
            cy1 = y1[sj:sj + 1, :]
            ca = area[sj:sj + 1, :]
            iw = jnp.maximum(jnp.minimum(rx1, cx1)
                             - jnp.maximum(rx0, cx0), 0.0)
            ih = jnp.maximum(jnp.minimum(ry1, cy1)
                             - jnp.maximum(ry0, cy0), 0.0)
            inter = iw * ih
            conf = inter > _T * (ra + ca - inter)
            m = conf & ((sj * L + lj_t) > (si * L + li_t))
            c_ref[si * L:(si + 1) * L, sj * L:(sj + 1) * L] = m.astype(bf16)

    def cond(c):
        alive, kept = c
        return jnp.sum(alive) > 0.0

    def rbody(c):
        alive, kept = c
        al8 = jnp.broadcast_to(alive, (8, 2048)).astype(bf16)
        ear = jnp.dot(al8, c_ref[...], preferred_element_type=f32)[0:1]
        frontier = (alive > 0.0) & (ear <= 0.5)
        fr8 = jnp.broadcast_to(frontier.astype(f32), (8, 2048)).astype(bf16)
        spread = jnp.dot(fr8, c_ref[...], preferred_element_type=f32)[0:1] > 0.5
        kept = jnp.where(frontier, 1.0, kept)
        alive = jnp.where(frontier | spread, 0.0, alive)
        return alive, kept

    alive0 = jnp.ones((1, 2048), f32)
    kept0 = jnp.zeros((1, 2048), f32)
    _, kept = lax.while_loop(cond, rbody, (alive0, kept0))

    l2 = lax.broadcasted_iota(i32, (1, 2048), 1)
    keptm = (kept > 0.0) & (l2 < _PRE)
    kv = keptm.astype(f32)
    run = kv
    for dd in (1, 2, 4, 8, 16, 32, 64, 128, 256, 512, 1024):
        run = run + jnp.where(l2 >= dd, pltpu.roll(run, dd, 1), 0.0)
    pos = run - kv

    o_io = lax.broadcasted_iota(i32, (L, 1024), 1).astype(f32)
    for si in range(16):
        ps = pos[0:1, si * L:(si + 1) * L].T
        ks = kv[0:1, si * L:(si + 1) * L].T
        pt_ref[si * L:(si + 1) * L, :] = \
            ((ps == o_io) & (ks > 0.5) & (ps < float(_POST))).astype(bf16)

    r16 = lax.broadcasted_iota(i32, (16, L), 0) * L \
        + lax.broadcasted_iota(i32, (16, L), 1)
    sc_s = jnp.where(r16 < _PRE, sc, 0.0)

    acc = jnp.zeros((16, 1024), f32)
    for si in range(16):
        dts = jnp.concatenate(
            [x0[si:si + 1], y0[si:si + 1], x1[si:si + 1], y1[si:si + 1],
             tx[si:si + 1], ty[si:si + 1], tw[si:si + 1], th[si:si + 1],
             sc_s[si:si + 1], jnp.zeros((7, L), f32)], axis=0)
        acc = acc + jnp.dot(dts, pt_ref[si * L:(si + 1) * L, :].astype(f32),
                            precision=lax.Precision.HIGHEST,
                            preferred_element_type=f32)
    out_ref[...] = acc


def _k3_call(gt_bbox, sc_c, ix_c, x0_c, y0_c, x1_c, y1_c):
    return pl.pallas_call(
        _k3_body,
        out_shape=jax.ShapeDtypeStruct((16, 1024), jnp.float32),
        in_specs=[
            pl.BlockSpec(memory_space=pltpu.SMEM),
            pl.BlockSpec(memory_space=pltpu.VMEM),
            pl.BlockSpec(memory_space=pltpu.VMEM),
            pl.BlockSpec(memory_space=pltpu.VMEM),
            pl.BlockSpec(memory_space=pltpu.VMEM),
            pl.BlockSpec(memory_space=pltpu.VMEM),
            pl.BlockSpec(memory_space=pltpu.VMEM),
        ],
        out_specs=pl.BlockSpec(memory_space=pltpu.VMEM),
        scratch_shapes=[
            pltpu.VMEM((2048, 2048), jnp.bfloat16),
            pltpu.VMEM((2048, 1024), jnp.bfloat16),
        ],
    )(gt_bbox, sc_c.reshape(32, 128), ix_c.reshape(32, 128),
      x0_c.reshape(32, 128), y0_c.reshape(32, 128),
      x1_c.reshape(32, 128), y1_c.reshape(32, 128))


def kernel(boxes, scores, gt_bbox):
    spf = jnp.concatenate(
        [scores, jnp.full((_PADN - _N,), _NEGI, jnp.float32)])
    sp = spf.reshape(_NB, 8, 128)
    k1 = _k1_call(sp)
    tau = k1[0, 0]
    needn = k1[1, 0]
    bx0 = jnp.concatenate([boxes[:, 0], jnp.zeros((_PADN - _N,), jnp.float32)])
    by0 = jnp.concatenate([boxes[:, 1], jnp.zeros((_PADN - _N,), jnp.float32)])
    bx1 = jnp.concatenate([boxes[:, 2], jnp.ones((_PADN - _N,), jnp.float32)])
    by1 = jnp.concatenate([boxes[:, 3], jnp.ones((_PADN - _N,), jnp.float32)])
    sc_c, ix_c, x0_c, y0_c, x1_c, y1_c = _k2_call(
        spf, bx0, by0, bx1, by1, tau, needn)
    out = _k3_call(gt_bbox, sc_c, ix_c, x0_c, y0_c, x1_c, y1_c)
    return out[:9, :_POST].T


# speedup vs baseline: 74.1986x; 1.0979x over previous
import functools

import jax
import jax.numpy as jnp
from jax import lax
from jax.experimental import pallas as pl
from jax.experimental.pallas import tpu as pltpu
from jax.experimental.pallas import tpu_sc as plsc

_N = 20000
_G = 20
_PRE = 2000
_POST = 1000
_T = 0.7
_NB = 20
_PADN = _NB * 1024
_CAP = 4096
_NEGI = float('-inf')


_SENT = 1 << 30


def _k1_body(sc_ref, tau_ref, need_ref):
    i32 = jnp.int32
    f32 = jnp.float32
    b = lax.bitcast_convert_type(sc_ref[...], i32)
    key = jnp.where(b < 0, (-2147483648) - b, b)
    cnt_nonneg = jnp.sum((key >= 0).astype(i32))
    neg = cnt_nonneg < _PRE
    lo0 = jnp.where(neg, -2139095041, -1)
    hi0 = jnp.where(neg, -1, 2139095039)

    def cond(st):
        lo, hi, fnd = st
        return (fnd == _SENT) & (lo < hi)

    def body(st):
        lo, hi, fnd = st
        mid = lo + (hi - lo + 1) // 2
        cg = jnp.sum((key > mid).astype(i32))
        ok = cg >= _PRE
        inw = ok & (cg <= _PRE + 64)
        fnd = jnp.where(inw, mid, fnd)
        lo = jnp.where(ok, mid, lo)
        hi = jnp.where(ok, hi, mid - 1)
        return lo, hi, fnd

    lo, _, fnd = lax.while_loop(cond, body, (lo0, hi0, _SENT))
    found = fnd != _SENT
    taukey = jnp.where(found, fnd, lo + 1)
    cg2 = jnp.sum((key > taukey).astype(i32))
    need = jnp.where(found, 0, _PRE - cg2)
    taubits = jnp.where(taukey >= 0, taukey, (-2147483648) - taukey)
    tau_ref[...] = lax.bitcast_convert_type(
        jnp.full((16,), taubits, i32), f32)
    need_ref[...] = jnp.full((16,), need, i32)


def _k1_call(sp):
    return pl.pallas_call(
        _k1_body,
        out_shape=[jax.ShapeDtypeStruct((16,), jnp.float32),
                   jax.ShapeDtypeStruct((16,), jnp.int32)],
        in_specs=[pl.BlockSpec(memory_space=pltpu.VMEM)],
        out_specs=[pl.BlockSpec(memory_space=pltpu.VMEM),
                   pl.BlockSpec(memory_space=pltpu.VMEM)],
    )(sp)


_STRIPE = 1280
_NTILES = 16
_RUP = 128


def _k2_body(sc_hbm, x0_hbm, y0_hbm, x1_hbm, y1_hbm, tau_hbm, need_hbm,
             osc, ox0, oy0, ox1, oy1, oix,
             sbuf, x0b, y0b, x1b, y1b,
             lsc, lx0, ly0, lx1, ly1, lix,
             taub, needb, cntb, allcnt, shared):
    i32 = jnp.int32
    f32 = jnp.float32
    c = lax.axis_index("c")
    w = lax.axis_index("s")

    @pl.when(c == 0)
    def _():
        base = pl.multiple_of(w * _STRIPE, 256)
        pltpu.sync_copy(sc_hbm.at[pl.ds(base, _STRIPE)], sbuf)
        pltpu.sync_copy(x0_hbm.at[pl.ds(base, _STRIPE)], x0b)
        pltpu.sync_copy(y0_hbm.at[pl.ds(base, _STRIPE)], y0b)
        pltpu.sync_copy(x1_hbm.at[pl.ds(base, _STRIPE)], x1b)
        pltpu.sync_copy(y1_hbm.at[pl.ds(base, _STRIPE)], y1b)
        pltpu.sync_copy(tau_hbm, taub)
        pltpu.sync_copy(need_hbm, needb)
        tauv = taub[...]
        needv = needb[...]
        lane = lax.broadcasted_iota(i32, (16,), 0)

        def pf(k, carry):
            lsc[pl.ds(k * 16, 16)] = jnp.full((16,), -jnp.inf, f32)
            lx0[pl.ds(k * 16, 16)] = jnp.full((16,), 1.0e7, f32)
            ly0[pl.ds(k * 16, 16)] = jnp.full((16,), 1.0e7, f32)
            lx1[pl.ds(k * 16, 16)] = jnp.full((16,), 1.0e7 + 1.0, f32)
            ly1[pl.ds(k * 16, 16)] = jnp.full((16,), 1.0e7 + 1.0, f32)
            lix[pl.ds(k * 16, 16)] = jnp.full((16,), 1 << 22, i32)
            return carry

        lax.fori_loop(0, 82, pf, 0)

        tb = pl.multiple_of(w * 256, 256)
        pltpu.sync_copy(lsc.at[pl.ds(0, 256)], osc.at[pl.ds(tb, 256)])
        pltpu.sync_copy(lx0.at[pl.ds(0, 256)], ox0.at[pl.ds(tb, 256)])
        pltpu.sync_copy(ly0.at[pl.ds(0, 256)], oy0.at[pl.ds(tb, 256)])
        pltpu.sync_copy(lx1.at[pl.ds(0, 256)], ox1.at[pl.ds(tb, 256)])
        pltpu.sync_copy(ly1.at[pl.ds(0, 256)], oy1.at[pl.ds(tb, 256)])
        pltpu.sync_copy(lix.at[pl.ds(0, 256)], oix.at[pl.ds(tb, 256)])

        def key_of(k):
            return sbuf[pl.ds(k * 16, 16)]

        def cnt(k, carry):
            cg, ce = carry
            key = key_of(k)
            cg = cg + plsc.all_reduce_population_count(key > tauv)
            ce = ce + plsc.all_reduce_population_count(key == tauv)
            return cg, ce

        z = jnp.zeros((16,), i32)
        cgv, cev = lax.fori_loop(0, 80, cnt, (z, z))
        cntb[...] = jnp.where(lane == 0, cgv, jnp.where(lane == 1, cev, 0))
        pltpu.sync_copy(cntb, shared.at[w])
        plsc.subcore_barrier()
        pltpu.sync_copy(shared, allcnt)

        needn = jnp.max(needv)

        def pfx(v, carry):
            eqp, off, myeqp, myoff = carry
            row = allcnt[v]
            cg = jnp.max(jnp.where(lane == 0, row, 0))
            ce = jnp.max(jnp.where(lane == 1, row, 0))
            n_v = cg + jnp.clip(needn - eqp, 0, ce)
            m_v = ((n_v + _RUP - 1) // _RUP) * _RUP
            myeqp = jnp.where(v == w, eqp, myeqp)
            myoff = jnp.where(v == w, off, myoff)
            return eqp + ce, off + m_v, myeqp, myoff

        zero = jnp.zeros((), i32)
        _, _, myeqp, myoff = lax.fori_loop(
            0, _NTILES, pfx, (zero, zero, zero, zero))

        def emit(k, carry):
            off, eqc = carry
            key = key_of(k)
            gtm = key > tauv
            eqm = key == tauv
            eqi = eqm.astype(i32)
            excl = plsc.cumsum(eqi) - eqi
            rank = myeqp + eqc + excl
            em = gtm | (eqm & (rank < needn))
            gidx = base + k * 16 + lane
            plsc.store_compressed(lsc.at[pl.ds(off, 16)],
                                  sbuf[pl.ds(k * 16, 16)], mask=em)
            plsc.store_compressed(lx0.at[pl.ds(off, 16)],
                                  x0b[pl.ds(k * 16, 16)], mask=em)
            plsc.store_compressed(ly0.at[pl.ds(off, 16)],
                                  y0b[pl.ds(k * 16, 16)], mask=em)
            plsc.store_compressed(lx1.at[pl.ds(off, 16)],
                                  x1b[pl.ds(k * 16, 16)], mask=em)
            plsc.store_compressed(ly1.at[pl.ds(off, 16)],
                                  y1b[pl.ds(k * 16, 16)], mask=em)
            plsc.store_compressed(lix.at[pl.ds(off, 16)], gidx, mask=em)
            npop = jnp.max(plsc.all_reduce_population_count(em))
            neq = jnp.max(plsc.all_reduce_population_count(eqm))
            return off + npop, eqc + neq

        n_w, _ = lax.fori_loop(0, 80, emit, (zero, zero))
        m_w = ((n_w + _RUP - 1) // _RUP) * _RUP

        def wr(g, carry):
            s = g * _RUP
            d = pl.multiple_of(myoff + g * _RUP, _RUP)
            pltpu.sync_copy(lsc.at[pl.ds(s, _RUP)], osc.at[pl.ds(d, _RUP)])
            pltpu.sync_copy(lx0.at[pl.ds(s, _RUP)], ox0.at[pl.ds(d, _RUP)])
            pltpu.sync_copy(ly0.at[pl.ds(s, _RUP)], oy0.at[pl.ds(d, _RUP)])
            pltpu.sync_copy(lx1.at[pl.ds(s, _RUP)], ox1.at[pl.ds(d, _RUP)])
            pltpu.sync_copy(ly1.at[pl.ds(s, _RUP)], oy1.at[pl.ds(d, _RUP)])
            pltpu.sync_copy(lix.at[pl.ds(s, _RUP)], oix.at[pl.ds(d, _RUP)])
            return carry

        lax.fori_loop(0, m_w // _RUP, wr, 0)


def _k2_call(spf, bx0, by0, bx1, by1, tau, needn):
    i32 = jnp.int32
    f32 = jnp.float32
    fn = functools.partial(
        pl.kernel,
        mesh=plsc.VectorSubcoreMesh(core_axis_name="c", subcore_axis_name="s"),
        compiler_params=pltpu.CompilerParams(needs_layout_passes=False),
        out_type=[
            jax.ShapeDtypeStruct((_CAP,), f32),
            jax.ShapeDtypeStruct((_CAP,), f32),
            jax.ShapeDtypeStruct((_CAP,), f32),
            jax.ShapeDtypeStruct((_CAP,), f32),
            jax.ShapeDtypeStruct((_CAP,), f32),
            jax.ShapeDtypeStruct((_CAP,), i32),
        ],
        scratch_types=[
            pltpu.VMEM((_STRIPE,), f32),
            pltpu.VMEM((_STRIPE,), f32),
            pltpu.VMEM((_STRIPE,), f32),
            pltpu.VMEM((_STRIPE,), f32),
            pltpu.VMEM((_STRIPE,), f32),
            pltpu.VMEM((1312,), f32),
            pltpu.VMEM((1312,), f32),
            pltpu.VMEM((1312,), f32),
            pltpu.VMEM((1312,), f32),
            pltpu.VMEM((1312,), f32),
            pltpu.VMEM((1312,), i32),
            pltpu.VMEM((16,), f32),
            pltpu.VMEM((16,), i32),
            pltpu.VMEM((16,), i32),
            pltpu.VMEM((16, 16), i32),
            pltpu.VMEM_SHARED((16, 16), i32),
        ],
    )(_k2_body)
    osc, ox0, oy0, ox1, oy1, oix = fn(spf, bx0, by0, bx1, by1, tau, needn)
    return osc, oix, ox0, oy0, ox1, oy1


def _k2_standin(spf, bx0, by0, bx1, by1, tau, needn):
    i32 = jnp.int32
    b = lax.bitcast_convert_type(spf, i32)
    key = jnp.where(b < 0, (-2147483648) - b, b)
    gt = key > tau
    eq = key == tau
    eqrank = jnp.cumsum(eq.astype(i32)) - 1
    emit = gt | (eq & (eqrank < needn))
    order = jnp.argsort(jnp.where(emit, jnp.arange(_PADN), 1 << 30))
    sel = order[:_CAP]
    valid = emit[sel]
    sc_c = jnp.where(valid, spf[sel], _NEGI)
    x0_c = jnp.where(valid, bx0[sel], 1.0e7)
    y0_c = jnp.where(valid, by0[sel], 1.0e7)
    x1_c = jnp.where(valid, bx1[sel], 1.0e7 + 1.0)
    y1_c = jnp.where(valid, by1[sel], 1.0e7 + 1.0)
    ix_c = jnp.where(valid, sel.astype(i32), (1 << 22))
    return sc_c, ix_c, x0_c, y0_c, x1_c, y1_c


def _k3_body(gt_ref, sc_ref, ix_ref, bx0_ref, by0_ref, bx1_ref, by1_ref,
             out_ref, c_ref, pt_ref):
    f32 = jnp.float32
    bf16 = jnp.bfloat16
    i32 = jnp.int32
    S, L = 32, 128
    s_io = lax.broadcasted_iota(i32, (S, L), 0)
    l_io = lax.broadcasted_iota(i32, (S, L), 1)
    f_io = s_io * L + l_io

    def xshuf(a, d):
        if d < L:
            fwd = pltpu.roll(a, L - d, 1)
            bwd = pltpu.roll(a, d, 1)
            bit = (l_io & d) == 0
        else:
            r = d // L
            fwd = pltpu.roll(a, S - r, 0)
            bwd = pltpu.roll(a, r, 0)
            bit = (s_io & r) == 0
        return jnp.where(bit, fwd, bwd)

    arrs = [sc_ref[...], ix_ref[...], bx0_ref[...], by0_ref[...],
            bx1_ref[...], by1_ref[...]]
    for kk in range(1, 13):
        size = 1 << kk
        for j in range(kk - 1, -1, -1):
            d = 1 << j
            p = [xshuf(a, d) for a in arrs]
            plt = (p[0] > arrs[0]) | ((p[0] == arrs[0]) & (p[1] < arrs[1]))
            lower = (f_io & d) == 0
            up = (f_io & size) == 0
            take = (lower == up) == plt
            arrs = [jnp.where(take, pa, a) for pa, a in zip(p, arrs)]
    sc = arrs[0][:16]
    x0 = arrs[2][:16]
    y0 = arrs[3][:16]
    x1 = arrs[4][:16]
    y1 = arrs[5][:16]

    area = (x1 - x0) * (y1 - y0)
    best = jnp.full((16, L), -1.0, f32)
    mg0 = jnp.zeros((16, L), f32)
    mg1 = jnp.zeros((16, L), f32)
    mg2 = jnp.ones((16, L), f32)
    mg3 = jnp.ones((16, L), f32)
    for g in range(_G):
        g0 = gt_ref[g, 0]
        g1 = gt_ref[g, 1]
        g2 = gt_ref[g, 2]
        g3 = gt_ref[g, 3]
        ga = (g2 - g0) * (g3 - g1)
        iw = jnp.maximum(jnp.minimum(x1, g2) - jnp.maximum(x0, g0), 0.0)
        ih = jnp.maximum(jnp.minimum(y1, g3) - jnp.maximum(y0, g1), 0.0)
        inter = iw * ih
        iou = inter / (area + ga - inter)
        upd = iou > best
        best = jnp.where(upd, iou, best)
        mg0 = jnp.where(upd, g0, mg0)
        mg1 = jnp.where(upd, g1, mg1)
        mg2 = jnp.where(upd, g2, mg2)
        mg3 = jnp.where(upd, g3, mg3)
    pw = x1 - x0
    ph = y1 - y0
    px = x0 + 0.5 * pw
    py = y0 + 0.5 * ph
    gw = mg2 - mg0
    gh = mg3 - mg1
    gx = mg0 + 0.5 * gw
    gy = mg1 + 0.5 * gh
    tx = (gx - px) / pw
    ty = (gy - py) / ph
    tw = jnp.log(gw / pw)
    th = jnp.log(gh / ph)

    c_ref[...] = jnp.zeros((2048, 2048), bf16)
    li_t = lax.broadcasted_iota(i32, (L, L), 0)
    lj_t = lax.broadcasted_iota(i32, (L, L), 1)
    sarea = area * (_T / (1.0 + _T))
    for si in range(16):
        rx0 = x0[si:si + 1, :].T
        ry0 = y0[si:si + 1, :].T
        rx1 = x1[si:si + 1, :].T
        ry1 = y1[si:si + 1, :].T
        ra = sarea[si:si + 1, :].T
        for sj in range(si, 16):
            iw = jnp.maximum(jnp.minimum(rx1, x1[sj:sj + 1, :])
                             - jnp.maximum(rx0, x0[sj:sj + 1, :]), 0.0)
            ih = jnp.maximum(jnp.minimum(ry1, y1[sj:sj + 1, :])
                             - jnp.maximum(ry0, y0[sj:sj + 1, :]), 0.0)
            conf = iw * ih > ra + sarea[sj:sj + 1, :]
            if si == sj:
                conf = conf & (lj_t > li_t)
            c_ref[si * L:(si + 1) * L, sj * L:(sj + 1) * L] = conf.astype(bf16)

    def cond(c):
        alive, kept = c
        return jnp.sum(alive) > 0.0

    def rbody(c):
        alive, kept = c
        al8 = jnp.broadcast_to(alive, (8, 2048)).astype(bf16)
        ear = jnp.dot(al8, c_ref[...], preferred_element_type=f32)[0:1]
        frontier = (alive > 0.0) & (ear <= 0.5)
        fr8 = jnp.broadcast_to(frontier.astype(f32), (8, 2048)).astype(bf16)
        spread = jnp.dot(fr8, c_ref[...], preferred_element_type=f32)[0:1] > 0.5
        kept = jnp.where(frontier, 1.0, kept)
        alive = jnp.where(frontier | spread, 0.0, alive)
        return alive, kept

    alive0 = jnp.ones((1, 2048), f32)
    kept0 = jnp.zeros((1, 2048), f32)
    _, kept = lax.while_loop(cond, rbody, (alive0, kept0))

    l2 = lax.broadcasted_iota(i32, (1, 2048), 1)
    keptm = (kept > 0.0) & (l2 < _PRE)
    kv = keptm.astype(f32)
    run = kv
    for dd in (1, 2, 4, 8, 16, 32, 64, 128, 256, 512, 1024):
        run = run + jnp.where(l2 >= dd, pltpu.roll(run, dd, 1), 0.0)
    pos = run - kv

    o_io = lax.broadcasted_iota(i32, (L, 1024), 1).astype(f32)
    for si in range(16):
        ps = pos[0:1, si * L:(si + 1) * L].T
        ks = kv[0:1, si * L:(si + 1) * L].T
        pt_ref[si * L:(si + 1) * L, :] = \
            ((ps == o_io) & (ks > 0.5) & (ps < float(_POST))).astype(bf16)

    r16 = lax.broadcasted_iota(i32, (16, L), 0) * L \
        + lax.broadcasted_iota(i32, (16, L), 1)
    sc_s = jnp.where(r16 < _PRE, sc, 0.0)

    acc = jnp.zeros((16, 1024), f32)
    for si in range(16):
        dts = jnp.concatenate(
            [x0[si:si + 1], y0[si:si + 1], x1[si:si + 1], y1[si:si + 1],
             tx[si:si + 1], ty[si:si + 1], tw[si:si + 1], th[si:si + 1],
             sc_s[si:si + 1], jnp.zeros((7, L), f32)], axis=0)
        acc = acc + jnp.dot(dts, pt_ref[si * L:(si + 1) * L, :].astype(f32),
                            precision=lax.Precision.HIGHEST,
                            preferred_element_type=f32)
    out_ref[...] = acc


def _k3_call(gt_bbox, sc_c, ix_c, x0_c, y0_c, x1_c, y1_c):
    return pl.pallas_call(
        _k3_body,
        out_shape=jax.ShapeDtypeStruct((16, 1024), jnp.float32),
        in_specs=[
            pl.BlockSpec(memory_space=pltpu.SMEM),
            pl.BlockSpec(memory_space=pltpu.VMEM),
            pl.BlockSpec(memory_space=pltpu.VMEM),
            pl.BlockSpec(memory_space=pltpu.VMEM),
            pl.BlockSpec(memory_space=pltpu.VMEM),
            pl.BlockSpec(memory_space=pltpu.VMEM),
            pl.BlockSpec(memory_space=pltpu.VMEM),
        ],
        out_specs=pl.BlockSpec(memory_space=pltpu.VMEM),
        scratch_shapes=[
            pltpu.VMEM((2048, 2048), jnp.bfloat16),
            pltpu.VMEM((2048, 1024), jnp.bfloat16),
        ],
    )(gt_bbox, sc_c.reshape(32, 128), ix_c.reshape(32, 128),
      x0_c.reshape(32, 128), y0_c.reshape(32, 128),
      x1_c.reshape(32, 128), y1_c.reshape(32, 128))


def kernel(boxes, scores, gt_bbox):
    spf = jnp.concatenate(
        [scores, jnp.full((_PADN - _N,), _NEGI, jnp.float32)])
    sp = spf.reshape(_NB, 8, 128)
    tau, needn = _k1_call(sp)
    bx0 = jnp.concatenate([boxes[:, 0], jnp.zeros((_PADN - _N,), jnp.float32)])
    by0 = jnp.concatenate([boxes[:, 1], jnp.zeros((_PADN - _N,), jnp.float32)])
    bx1 = jnp.concatenate([boxes[:, 2], jnp.ones((_PADN - _N,), jnp.float32)])
    by1 = jnp.concatenate([boxes[:, 3], jnp.ones((_PADN - _N,), jnp.float32)])
    sc_c, ix_c, x0_c, y0_c, x1_c, y1_c = _k2_call(
        spf, bx0, by0, bx1, by1, tau, needn)
    out = _k3_call(gt_bbox, sc_c, ix_c, x0_c, y0_c, x1_c, y1_c)
    return out[:9, :_POST].T


# speedup vs baseline: 77.4926x; 1.0444x over previous
import functools

import jax
import jax.numpy as jnp
from jax import lax
from jax.experimental import pallas as pl
from jax.experimental.pallas import tpu as pltpu
from jax.experimental.pallas import tpu_sc as plsc

_N = 20000
_G = 20
_PRE = 2000
_POST = 1000
_T = 0.7
_NB = 20
_PADN = _NB * 1024
_CAP = 4096
_NEGI = float('-inf')


_SENT = 1 << 30


def _k1_body(sc_ref, tau_ref, need_ref):
    i32 = jnp.int32
    f32 = jnp.float32
    b = lax.bitcast_convert_type(sc_ref[...], i32)
    key = jnp.where(b < 0, (-2147483648) - b, b)
    cnt_nonneg = jnp.sum((key >= 0).astype(i32))
    neg = cnt_nonneg < _PRE
    lo0 = jnp.where(neg, -2139095041, -1)
    hi0 = jnp.where(neg, -1, 2139095039)

    def cond(st):
        lo, hi, fnd = st
        return (fnd == _SENT) & (lo < hi)

    def body(st):
        lo, hi, fnd = st
        mid = lo + (hi - lo + 1) // 2
        cg = jnp.sum((key > mid).astype(i32))
        ok = cg >= _PRE
        inw = ok & (cg <= _PRE + 64)
        fnd = jnp.where(inw, mid, fnd)
        lo = jnp.where(ok, mid, lo)
        hi = jnp.where(ok, hi, mid - 1)
        return lo, hi, fnd

    lo, _, fnd = lax.while_loop(cond, body, (lo0, hi0, _SENT))
    found = fnd != _SENT
    taukey = jnp.where(found, fnd, lo + 1)
    cg2 = jnp.sum((key > taukey).astype(i32))
    need = jnp.where(found, 0, _PRE - cg2)
    taubits = jnp.where(taukey >= 0, taukey, (-2147483648) - taukey)
    tau_ref[...] = lax.bitcast_convert_type(
        jnp.full((16,), taubits, i32), f32)
    need_ref[...] = jnp.full((16,), need, i32)


def _k1_call(sp):
    return pl.pallas_call(
        _k1_body,
        out_shape=[jax.ShapeDtypeStruct((16,), jnp.float32),
                   jax.ShapeDtypeStruct((16,), jnp.int32)],
        in_specs=[pl.BlockSpec(memory_space=pltpu.VMEM)],
        out_specs=[pl.BlockSpec(memory_space=pltpu.VMEM),
                   pl.BlockSpec(memory_space=pltpu.VMEM)],
    )(sp)


_STRIPE = 1280
_NTILES = 16
_RUP = 128


def _k2_body(sc_hbm, x0_hbm, y0_hbm, x1_hbm, y1_hbm, tau_hbm, need_hbm,
             osc, ox0, oy0, ox1, oy1, oix,
             sbuf, x0b, y0b, x1b, y1b,
             lsc, lx0, ly0, lx1, ly1, lix,
             taub, needb, cntb, allcnt, shared):
    i32 = jnp.int32
    f32 = jnp.float32
    c = lax.axis_index("c")
    w = lax.axis_index("s")

    @pl.when(c == 0)
    def _():
        base = pl.multiple_of(w * _STRIPE, 256)
        pltpu.sync_copy(sc_hbm.at[pl.ds(base, _STRIPE)], sbuf)
        pltpu.sync_copy(x0_hbm.at[pl.ds(base, _STRIPE)], x0b)
        pltpu.sync_copy(y0_hbm.at[pl.ds(base, _STRIPE)], y0b)
        pltpu.sync_copy(x1_hbm.at[pl.ds(base, _STRIPE)], x1b)
        pltpu.sync_copy(y1_hbm.at[pl.ds(base, _STRIPE)], y1b)
        pltpu.sync_copy(tau_hbm, taub)
        pltpu.sync_copy(need_hbm, needb)
        tauv = taub[...]
        needv = needb[...]
        lane = lax.broadcasted_iota(i32, (16,), 0)

        def pf(k, carry):
            lsc[pl.ds(k * 16, 16)] = jnp.full((16,), -jnp.inf, f32)
            lx0[pl.ds(k * 16, 16)] = jnp.full((16,), 1.0e7, f32)
            ly0[pl.ds(k * 16, 16)] = jnp.full((16,), 1.0e7, f32)
            lx1[pl.ds(k * 16, 16)] = jnp.full((16,), 1.0e7 + 1.0, f32)
            ly1[pl.ds(k * 16, 16)] = jnp.full((16,), 1.0e7 + 1.0, f32)
            lix[pl.ds(k * 16, 16)] = jnp.full((16,), 1 << 22, i32)
            return carry

        lax.fori_loop(0, 82, pf, 0)

        tb = pl.multiple_of(w * 256, 256)
        pltpu.sync_copy(lsc.at[pl.ds(0, 256)], osc.at[pl.ds(tb, 256)])
        pltpu.sync_copy(lx0.at[pl.ds(0, 256)], ox0.at[pl.ds(tb, 256)])
        pltpu.sync_copy(ly0.at[pl.ds(0, 256)], oy0.at[pl.ds(tb, 256)])
        pltpu.sync_copy(lx1.at[pl.ds(0, 256)], ox1.at[pl.ds(tb, 256)])
        pltpu.sync_copy(ly1.at[pl.ds(0, 256)], oy1.at[pl.ds(tb, 256)])
        pltpu.sync_copy(lix.at[pl.ds(0, 256)], oix.at[pl.ds(tb, 256)])

        def key_of(k):
            return sbuf[pl.ds(k * 16, 16)]

        def cnt(k, carry):
            cg, ce = carry
            key = key_of(k)
            cg = cg + plsc.all_reduce_population_count(key > tauv)
            ce = ce + plsc.all_reduce_population_count(key == tauv)
            return cg, ce

        z = jnp.zeros((16,), i32)
        cgv, cev = lax.fori_loop(0, 80, cnt, (z, z))
        cntb[...] = jnp.where(lane == 0, cgv, jnp.where(lane == 1, cev, 0))
        pltpu.sync_copy(cntb, shared.at[w])
        plsc.subcore_barrier()
        pltpu.sync_copy(shared, allcnt)

        needn = jnp.max(needv)

        def pfx(v, carry):
            eqp, off, myeqp, myoff = carry
            row = allcnt[v]
            cg = jnp.max(jnp.where(lane == 0, row, 0))
            ce = jnp.max(jnp.where(lane == 1, row, 0))
            n_v = cg + jnp.clip(needn - eqp, 0, ce)
            m_v = ((n_v + _RUP - 1) // _RUP) * _RUP
            myeqp = jnp.where(v == w, eqp, myeqp)
            myoff = jnp.where(v == w, off, myoff)
            return eqp + ce, off + m_v, myeqp, myoff

        zero = jnp.zeros((), i32)
        _, _, myeqp, myoff = lax.fori_loop(
            0, _NTILES, pfx, (zero, zero, zero, zero))

        def emit(k, carry):
            off, eqc = carry
            key = key_of(k)
            gtm = key > tauv
            eqm = key == tauv
            eqi = eqm.astype(i32)
            excl = plsc.cumsum(eqi) - eqi
            rank = myeqp + eqc + excl
            em = gtm | (eqm & (rank < needn))
            gidx = base + k * 16 + lane
            plsc.store_compressed(lsc.at[pl.ds(off, 16)],
                                  sbuf[pl.ds(k * 16, 16)], mask=em)
            plsc.store_compressed(lx0.at[pl.ds(off, 16)],
                                  x0b[pl.ds(k * 16, 16)], mask=em)
            plsc.store_compressed(ly0.at[pl.ds(off, 16)],
                                  y0b[pl.ds(k * 16, 16)], mask=em)
            plsc.store_compressed(lx1.at[pl.ds(off, 16)],
                                  x1b[pl.ds(k * 16, 16)], mask=em)
            plsc.store_compressed(ly1.at[pl.ds(off, 16)],
                                  y1b[pl.ds(k * 16, 16)], mask=em)
            plsc.store_compressed(lix.at[pl.ds(off, 16)], gidx, mask=em)
            npop = jnp.max(plsc.all_reduce_population_count(em))
            neq = jnp.max(plsc.all_reduce_population_count(eqm))
            return off + npop, eqc + neq

        n_w, _ = lax.fori_loop(0, 80, emit, (zero, zero))
        m_w = ((n_w + _RUP - 1) // _RUP) * _RUP

        def wr(g, carry):
            s = g * _RUP
            d = pl.multiple_of(myoff + g * _RUP, _RUP)
            pltpu.sync_copy(lsc.at[pl.ds(s, _RUP)], osc.at[pl.ds(d, _RUP)])
            pltpu.sync_copy(lx0.at[pl.ds(s, _RUP)], ox0.at[pl.ds(d, _RUP)])
            pltpu.sync_copy(ly0.at[pl.ds(s, _RUP)], oy0.at[pl.ds(d, _RUP)])
            pltpu.sync_copy(lx1.at[pl.ds(s, _RUP)], ox1.at[pl.ds(d, _RUP)])
            pltpu.sync_copy(ly1.at[pl.ds(s, _RUP)], oy1.at[pl.ds(d, _RUP)])
            pltpu.sync_copy(lix.at[pl.ds(s, _RUP)], oix.at[pl.ds(d, _RUP)])
            return carry

        lax.fori_loop(0, m_w // _RUP, wr, 0)


def _k2_call(spf, bx0, by0, bx1, by1, tau, needn):
    i32 = jnp.int32
    f32 = jnp.float32
    fn = functools.partial(
        pl.kernel,
        mesh=plsc.VectorSubcoreMesh(core_axis_name="c", subcore_axis_name="s"),
        compiler_params=pltpu.CompilerParams(needs_layout_passes=False),
        out_type=[
            jax.ShapeDtypeStruct((_CAP,), f32),
            jax.ShapeDtypeStruct((_CAP,), f32),
            jax.ShapeDtypeStruct((_CAP,), f32),
            jax.ShapeDtypeStruct((_CAP,), f32),
            jax.ShapeDtypeStruct((_CAP,), f32),
            jax.ShapeDtypeStruct((_CAP,), i32),
        ],
        scratch_types=[
            pltpu.VMEM((_STRIPE,), f32),
            pltpu.VMEM((_STRIPE,), f32),
            pltpu.VMEM((_STRIPE,), f32),
            pltpu.VMEM((_STRIPE,), f32),
            pltpu.VMEM((_STRIPE,), f32),
            pltpu.VMEM((1312,), f32),
            pltpu.VMEM((1312,), f32),
            pltpu.VMEM((1312,), f32),
            pltpu.VMEM((1312,), f32),
            pltpu.VMEM((1312,), f32),
            pltpu.VMEM((1312,), i32),
            pltpu.VMEM((16,), f32),
            pltpu.VMEM((16,), i32),
            pltpu.VMEM((16,), i32),
            pltpu.VMEM((16, 16), i32),
            pltpu.VMEM_SHARED((16, 16), i32),
        ],
    )(_k2_body)
    osc, ox0, oy0, ox1, oy1, oix = fn(spf, bx0, by0, bx1, by1, tau, needn)
    return osc, oix, ox0, oy0, ox1, oy1


def _k3_body(gt_ref, sc_ref, ix_ref, bx0_ref, by0_ref, bx1_ref, by1_ref,
             out_ref, c_ref, pt_ref):
    f32 = jnp.float32
    bf16 = jnp.bfloat16
    i32 = jnp.int32
    S, L = 32, 128
    s_io = lax.broadcasted_iota(i32, (S, L), 0)
    l_io = lax.broadcasted_iota(i32, (S, L), 1)
    f_io = s_io * L + l_io

    def xshuf(a, d):
        if d < L:
            fwd = pltpu.roll(a, L - d, 1)
            bwd = pltpu.roll(a, d, 1)
            bit = (l_io & d) == 0
        else:
            r = d // L
            fwd = pltpu.roll(a, S - r, 0)
            bwd = pltpu.roll(a, r, 0)
            bit = (s_io & r) == 0
        return jnp.where(bit, fwd, bwd)

    arrs = [sc_ref[...], ix_ref[...], bx0_ref[...], by0_ref[...],
            bx1_ref[...], by1_ref[...]]
    for kk in range(1, 13):
        size = 1 << kk
        for j in range(kk - 1, -1, -1):
            d = 1 << j
            p = [xshuf(a, d) for a in arrs]
            plt = (p[0] > arrs[0]) | ((p[0] == arrs[0]) & (p[1] < arrs[1]))
            lower = (f_io & d) == 0
            up = (f_io & size) == 0
            take = (lower == up) == plt
            arrs = [jnp.where(take, pa, a) for pa, a in zip(p, arrs)]
    sc = arrs[0][:16]
    x0 = arrs[2][:16]
    y0 = arrs[3][:16]
    x1 = arrs[4][:16]
    y1 = arrs[5][:16]

    area = (x1 - x0) * (y1 - y0)
    best = jnp.full((16, L), -1.0, f32)
    mg0 = jnp.zeros((16, L), f32)
    mg1 = jnp.zeros((16, L), f32)
    mg2 = jnp.ones((16, L), f32)
    mg3 = jnp.ones((16, L), f32)
    for g in range(_G):
        g0 = gt_ref[g, 0]
        g1 = gt_ref[g, 1]
        g2 = gt_ref[g, 2]
        g3 = gt_ref[g, 3]
        ga = (g2 - g0) * (g3 - g1)
        iw = jnp.maximum(jnp.minimum(x1, g2) - jnp.maximum(x0, g0), 0.0)
        ih = jnp.maximum(jnp.minimum(y1, g3) - jnp.maximum(y0, g1), 0.0)
        inter = iw * ih
        iou = inter / (area + ga - inter)
        upd = iou > best
        best = jnp.where(upd, iou, best)
        mg0 = jnp.where(upd, g0, mg0)
        mg1 = jnp.where(upd, g1, mg1)
        mg2 = jnp.where(upd, g2, mg2)
        mg3 = jnp.where(upd, g3, mg3)
    pw = x1 - x0
    ph = y1 - y0
    px = x0 + 0.5 * pw
    py = y0 + 0.5 * ph
    gw = mg2 - mg0
    gh = mg3 - mg1
    gx = mg0 + 0.5 * gw
    gy = mg1 + 0.5 * gh
    tx = (gx - px) / pw
    ty = (gy - py) / ph
    tw = jnp.log(gw / pw)
    th = jnp.log(gh / ph)

    c_ref[...] = jnp.zeros((2048, 2048), bf16)
    li_t = lax.broadcasted_iota(i32, (L, L), 0)
    lj_t = lax.broadcasted_iota(i32, (L, L), 1)
    sarea = area * (_T / (1.0 + _T))
    for si in range(16):
        rx0 = x0[si:si + 1, :].T
        ry0 = y0[si:si + 1, :].T
        rx1 = x1[si:si + 1, :].T
        ry1 = y1[si:si + 1, :].T
        ra = sarea[si:si + 1, :].T
        for sj in range(si, 16):
            iw = jnp.maximum(jnp.minimum(rx1, x1[sj:sj + 1, :])
                             - jnp.maximum(rx0, x0[sj:sj + 1, :]), 0.0)
            ih = jnp.maximum(jnp.minimum(ry1, y1[sj:sj + 1, :])
                             - jnp.maximum(ry0, y0[sj:sj + 1, :]), 0.0)
            conf = iw * ih > ra + sarea[sj:sj + 1, :]
            if si == sj:
                conf = conf & (lj_t > li_t)
            c_ref[si * L:(si + 1) * L, sj * L:(sj + 1) * L] = conf.astype(bf16)

    def cond(c):
        alive, kept = c
        return jnp.sum(alive) > 0.0

    def rbody(c):
        alive, kept = c
        al8 = jnp.broadcast_to(alive, (8, 2048)).astype(bf16)
        ear = jnp.dot(al8, c_ref[...], preferred_element_type=f32)[0:1]
        frontier = (alive > 0.0) & (ear <= 0.5)
        fr8 = jnp.broadcast_to(frontier.astype(f32), (8, 2048)).astype(bf16)
        spread = jnp.dot(fr8, c_ref[...], preferred_element_type=f32)[0:1] > 0.5
        kept = jnp.where(frontier, 1.0, kept)
        alive = jnp.where(frontier | spread, 0.0, alive)
        return alive, kept

    alive0 = jnp.ones((1, 2048), f32)
    kept0 = jnp.zeros((1, 2048), f32)
    _, kept = lax.while_loop(cond, rbody, (alive0, kept0))

    l2 = lax.broadcasted_iota(i32, (1, 2048), 1)
    keptm = (kept > 0.0) & (l2 < _PRE)
    kv = keptm.astype(f32)
    run = kv
    for dd in (1, 2, 4, 8, 16, 32, 64, 128, 256, 512, 1024):
        run = run + jnp.where(l2 >= dd, pltpu.roll(run, dd, 1), 0.0)
    pos = run - kv

    o_io = lax.broadcasted_iota(i32, (L, 1024), 1).astype(f32)
    for si in range(16):
        ps = pos[0:1, si * L:(si + 1) * L].T
        ks = kv[0:1, si * L:(si + 1) * L].T
        pt_ref[si * L:(si + 1) * L, :] = \
            ((ps == o_io) & (ks > 0.5) & (ps < float(_POST))).astype(bf16)

    r16 = lax.broadcasted_iota(i32, (16, L), 0) * L \
        + lax.broadcasted_iota(i32, (16, L), 1)
    sc_s = jnp.where(r16 < _PRE, sc, 0.0)

    acc = jnp.zeros((16, 1024), f32)
    for si in range(16):
        dts = jnp.concatenate(
            [x0[si:si + 1], y0[si:si + 1], x1[si:si + 1], y1[si:si + 1],
             tx[si:si + 1], ty[si:si + 1], tw[si:si + 1], th[si:si + 1],
             sc_s[si:si + 1], jnp.zeros((7, L), f32)], axis=0)
        acc = acc + jnp.dot(dts, pt_ref[si * L:(si + 1) * L, :].astype(f32),
                            precision=lax.Precision.HIGHEST,
                            preferred_element_type=f32)
    out_ref[...] = acc


def _k3_call(gt_bbox, sc_c, ix_c, x0_c, y0_c, x1_c, y1_c):
    return pl.pallas_call(
        _k3_body,
        out_shape=jax.ShapeDtypeStruct((16, 1024), jnp.float32),
        in_specs=[
            pl.BlockSpec(memory_space=pltpu.SMEM),
            pl.BlockSpec(memory_space=pltpu.VMEM),
            pl.BlockSpec(memory_space=pltpu.VMEM),
            pl.BlockSpec(memory_space=pltpu.VMEM),
            pl.BlockSpec(memory_space=pltpu.VMEM),
            pl.BlockSpec(memory_space=pltpu.VMEM),
            pl.BlockSpec(memory_space=pltpu.VMEM),
        ],
        out_specs=pl.BlockSpec(memory_space=pltpu.VMEM),
        scratch_shapes=[
            pltpu.VMEM((2048, 2048), jnp.bfloat16),
            pltpu.VMEM((2048, 1024), jnp.bfloat16),
        ],
    )(gt_bbox, sc_c.reshape(32, 128), ix_c.reshape(32, 128),
      x0_c.reshape(32, 128), y0_c.reshape(32, 128),
      x1_c.reshape(32, 128), y1_c.reshape(32, 128))


def kernel(boxes, scores, gt_bbox):
    spf = jnp.concatenate(
        [scores, jnp.full((_PADN - _N,), _NEGI, jnp.float32)])
    sp = spf.reshape(_NB, 8, 128)
    tau, needn = _k1_call(sp)
    padb = jnp.broadcast_to(jnp.array([0.0, 0.0, 1.0, 1.0], jnp.float32),
                            (_PADN - _N, 4))
    cols = jnp.concatenate([boxes, padb], axis=0).T
    bx0, by0, bx1, by1 = cols[0], cols[1], cols[2], cols[3]
    sc_c, ix_c, x0_c, y0_c, x1_c, y1_c = _k2_call(
        spf, bx0, by0, bx1, by1, tau, needn)
    out = _k3_call(gt_bbox, sc_c, ix_c, x0_c, y0_c, x1_c, y1_c)
    return out[:9, :_POST].T


# speedup vs baseline: 77.9903x; 1.0064x over previous
import functools

import jax
import jax.numpy as jnp
from jax import lax
from jax.experimental import pallas as pl
from jax.experimental.pallas import tpu as pltpu
from jax.experimental.pallas import tpu_sc as plsc

_N = 20000
_G = 20
_PRE = 2000
_POST = 1000
_T = 0.7
_NB = 20
_PADN = _NB * 1024
_CAP = 4096
_NEGI = float('-inf')


_SENT = 1 << 30


def _k1_body(sc_ref, tau_ref, need_ref):
    i32 = jnp.int32
    f32 = jnp.float32
    b = lax.bitcast_convert_type(sc_ref[...], i32)
    key = jnp.where(b < 0, (-2147483648) - b, b)
    cnt_nonneg = jnp.sum((key >= 0).astype(i32))
    neg = cnt_nonneg < _PRE
    lo0 = jnp.where(neg, -2139095041, -1)
    hi0 = jnp.where(neg, -1, 2139095039)

    def cond(st):
        lo, hi, fnd = st
        return (fnd == _SENT) & (lo < hi)

    def body(st):
        lo, hi, fnd = st
        mid = lo + (hi - lo + 1) // 2
        cg = jnp.sum((key > mid).astype(i32))
        ok = cg >= _PRE
        inw = ok & (cg <= _PRE + 64)
        fnd = jnp.where(inw, mid, fnd)
        lo = jnp.where(ok, mid, lo)
        hi = jnp.where(ok, hi, mid - 1)
        return lo, hi, fnd

    lo, _, fnd = lax.while_loop(cond, body, (lo0, hi0, _SENT))
    found = fnd != _SENT
    taukey = jnp.where(found, fnd, lo + 1)
    cg2 = jnp.sum((key > taukey).astype(i32))
    need = jnp.where(found, 0, _PRE - cg2)
    taubits = jnp.where(taukey >= 0, taukey, (-2147483648) - taukey)
    tau_ref[...] = lax.bitcast_convert_type(
        jnp.full((16,), taubits, i32), f32)
    need_ref[...] = jnp.full((16,), need, i32)


def _k1_call(sp):
    return pl.pallas_call(
        _k1_body,
        out_shape=[jax.ShapeDtypeStruct((16,), jnp.float32),
                   jax.ShapeDtypeStruct((16,), jnp.int32)],
        in_specs=[pl.BlockSpec(memory_space=pltpu.VMEM)],
        out_specs=[pl.BlockSpec(memory_space=pltpu.VMEM),
                   pl.BlockSpec(memory_space=pltpu.VMEM)],
    )(sp)


_STRIPE = 1280
_NTILES = 16
_RUP = 128


def _k2_body(sc_hbm, x0_hbm, y0_hbm, x1_hbm, y1_hbm, tau_hbm, need_hbm,
             osc, ox0, oy0, ox1, oy1, oix,
             sbuf, x0b, y0b, x1b, y1b,
             lsc, lx0, ly0, lx1, ly1, lix,
             taub, needb, cntb, allcnt, shared):
    i32 = jnp.int32
    f32 = jnp.float32
    c = lax.axis_index("c")
    w = lax.axis_index("s")

    @pl.when(c == 0)
    def _():
        base = pl.multiple_of(w * _STRIPE, 256)
        pltpu.sync_copy(sc_hbm.at[pl.ds(base, _STRIPE)], sbuf)
        pltpu.sync_copy(x0_hbm.at[pl.ds(base, _STRIPE)], x0b)
        pltpu.sync_copy(y0_hbm.at[pl.ds(base, _STRIPE)], y0b)
        pltpu.sync_copy(x1_hbm.at[pl.ds(base, _STRIPE)], x1b)
        pltpu.sync_copy(y1_hbm.at[pl.ds(base, _STRIPE)], y1b)
        pltpu.sync_copy(tau_hbm, taub)
        pltpu.sync_copy(need_hbm, needb)
        tauv = taub[...]
        needv = needb[...]
        lane = lax.broadcasted_iota(i32, (16,), 0)

        def pf(k, carry):
            lsc[pl.ds(k * 16, 16)] = jnp.full((16,), -jnp.inf, f32)
            lx0[pl.ds(k * 16, 16)] = jnp.full((16,), 1.0e7, f32)
            ly0[pl.ds(k * 16, 16)] = jnp.full((16,), 1.0e7, f32)
            lx1[pl.ds(k * 16, 16)] = jnp.full((16,), 1.0e7 + 1.0, f32)
            ly1[pl.ds(k * 16, 16)] = jnp.full((16,), 1.0e7 + 1.0, f32)
            lix[pl.ds(k * 16, 16)] = jnp.full((16,), 1 << 22, i32)
            return carry

        lax.fori_loop(0, 82, pf, 0)

        tb = pl.multiple_of(w * 256, 256)
        pltpu.sync_copy(lsc.at[pl.ds(0, 256)], osc.at[pl.ds(tb, 256)])
        pltpu.sync_copy(lx0.at[pl.ds(0, 256)], ox0.at[pl.ds(tb, 256)])
        pltpu.sync_copy(ly0.at[pl.ds(0, 256)], oy0.at[pl.ds(tb, 256)])
        pltpu.sync_copy(lx1.at[pl.ds(0, 256)], ox1.at[pl.ds(tb, 256)])
        pltpu.sync_copy(ly1.at[pl.ds(0, 256)], oy1.at[pl.ds(tb, 256)])
        pltpu.sync_copy(lix.at[pl.ds(0, 256)], oix.at[pl.ds(tb, 256)])

        def key_of(k):
            return sbuf[pl.ds(k * 16, 16)]

        def cnt(k, carry):
            cg, ce = carry
            key = key_of(k)
            cg = cg + plsc.all_reduce_population_count(key > tauv)
            ce = ce + plsc.all_reduce_population_count(key == tauv)
            return cg, ce

        z = jnp.zeros((16,), i32)
        cgv, cev = lax.fori_loop(0, 80, cnt, (z, z))
        cntb[...] = jnp.where(lane == 0, cgv, jnp.where(lane == 1, cev, 0))
        pltpu.sync_copy(cntb, shared.at[w])
        plsc.subcore_barrier()
        pltpu.sync_copy(shared, allcnt)

        needn = jnp.max(needv)

        def pfx(v, carry):
            eqp, off, myeqp, myoff = carry
            row = allcnt[v]
            cg = jnp.max(jnp.where(lane == 0, row, 0))
            ce = jnp.max(jnp.where(lane == 1, row, 0))
            n_v = cg + jnp.clip(needn - eqp, 0, ce)
            m_v = ((n_v + _RUP - 1) // _RUP) * _RUP
            myeqp = jnp.where(v == w, eqp, myeqp)
            myoff = jnp.where(v == w, off, myoff)
            return eqp + ce, off + m_v, myeqp, myoff

        zero = jnp.zeros((), i32)
        _, _, myeqp, myoff = lax.fori_loop(
            0, _NTILES, pfx, (zero, zero, zero, zero))

        def emit(k, carry):
            off, eqc = carry
            key = key_of(k)
            gtm = key > tauv
            eqm = key == tauv
            eqi = eqm.astype(i32)
            excl = plsc.cumsum(eqi) - eqi
            rank = myeqp + eqc + excl
            em = gtm | (eqm & (rank < needn))
            gidx = base + k * 16 + lane
            plsc.store_compressed(lsc.at[pl.ds(off, 16)],
                                  sbuf[pl.ds(k * 16, 16)], mask=em)
            plsc.store_compressed(lx0.at[pl.ds(off, 16)],
                                  x0b[pl.ds(k * 16, 16)], mask=em)
            plsc.store_compressed(ly0.at[pl.ds(off, 16)],
                                  y0b[pl.ds(k * 16, 16)], mask=em)
            plsc.store_compressed(lx1.at[pl.ds(off, 16)],
                                  x1b[pl.ds(k * 16, 16)], mask=em)
            plsc.store_compressed(ly1.at[pl.ds(off, 16)],
                                  y1b[pl.ds(k * 16, 16)], mask=em)
            plsc.store_compressed(lix.at[pl.ds(off, 16)], gidx, mask=em)
            npop = jnp.max(plsc.all_reduce_population_count(em))
            neq = jnp.max(plsc.all_reduce_population_count(eqm))
            return off + npop, eqc + neq

        n_w, _ = lax.fori_loop(0, 80, emit, (zero, zero))
        m_w = ((n_w + _RUP - 1) // _RUP) * _RUP

        def wr(g, carry):
            s = g * _RUP
            d = pl.multiple_of(myoff + g * _RUP, _RUP)
            pltpu.sync_copy(lsc.at[pl.ds(s, _RUP)], osc.at[pl.ds(d, _RUP)])
            pltpu.sync_copy(lx0.at[pl.ds(s, _RUP)], ox0.at[pl.ds(d, _RUP)])
            pltpu.sync_copy(ly0.at[pl.ds(s, _RUP)], oy0.at[pl.ds(d, _RUP)])
            pltpu.sync_copy(lx1.at[pl.ds(s, _RUP)], ox1.at[pl.ds(d, _RUP)])
            pltpu.sync_copy(ly1.at[pl.ds(s, _RUP)], oy1.at[pl.ds(d, _RUP)])
            pltpu.sync_copy(lix.at[pl.ds(s, _RUP)], oix.at[pl.ds(d, _RUP)])
            return carry

        lax.fori_loop(0, m_w // _RUP, wr, 0)


def _k2_call(spf, bx0, by0, bx1, by1, tau, needn):
    i32 = jnp.int32
    f32 = jnp.float32
    fn = functools.partial(
        pl.kernel,
        mesh=plsc.VectorSubcoreMesh(core_axis_name="c", subcore_axis_name="s"),
        compiler_params=pltpu.CompilerParams(needs_layout_passes=False),
        out_type=[
            jax.ShapeDtypeStruct((_CAP,), f32),
            jax.ShapeDtypeStruct((_CAP,), f32),
            jax.ShapeDtypeStruct((_CAP,), f32),
            jax.ShapeDtypeStruct((_CAP,), f32),
            jax.ShapeDtypeStruct((_CAP,), f32),
            jax.ShapeDtypeStruct((_CAP,), i32),
        ],
        scratch_types=[
            pltpu.VMEM((_STRIPE,), f32),
            pltpu.VMEM((_STRIPE,), f32),
            pltpu.VMEM((_STRIPE,), f32),
            pltpu.VMEM((_STRIPE,), f32),
            pltpu.VMEM((_STRIPE,), f32),
            pltpu.VMEM((1312,), f32),
            pltpu.VMEM((1312,), f32),
            pltpu.VMEM((1312,), f32),
            pltpu.VMEM((1312,), f32),
            pltpu.VMEM((1312,), f32),
            pltpu.VMEM((1312,), i32),
            pltpu.VMEM((16,), f32),
            pltpu.VMEM((16,), i32),
            pltpu.VMEM((16,), i32),
            pltpu.VMEM((16, 16), i32),
            pltpu.VMEM_SHARED((16, 16), i32),
        ],
    )(_k2_body)
    osc, ox0, oy0, ox1, oy1, oix = fn(spf, bx0, by0, bx1, by1, tau, needn)
    return osc, oix, ox0, oy0, ox1, oy1


def _k3_body(gt_ref, sc_ref, ix_ref, bx0_ref, by0_ref, bx1_ref, by1_ref,
             out_ref, c_ref, pt_ref):
    f32 = jnp.float32
    bf16 = jnp.bfloat16
    i32 = jnp.int32
    S, L = 32, 128
    s_io = lax.broadcasted_iota(i32, (S, L), 0)
    l_io = lax.broadcasted_iota(i32, (S, L), 1)
    f_io = s_io * L + l_io

    def xshuf(a, d):
        if d < L:
            fwd = pltpu.roll(a, L - d, 1)
            bwd = pltpu.roll(a, d, 1)
            bit = (l_io & d) == 0
        else:
            r = d // L
            fwd = pltpu.roll(a, S - r, 0)
            bwd = pltpu.roll(a, r, 0)
            bit = (s_io & r) == 0
        return jnp.where(bit, fwd, bwd)

    arrs = [sc_ref[...], ix_ref[...], bx0_ref[...], by0_ref[...],
            bx1_ref[...], by1_ref[...]]
    for kk in range(1, 13):
        size = 1 << kk
        for j in range(kk - 1, -1, -1):
            d = 1 << j
            p = [xshuf(a, d) for a in arrs]
            plt = (p[0] > arrs[0]) | ((p[0] == arrs[0]) & (p[1] < arrs[1]))
            lower = (f_io & d) == 0
            up = (f_io & size) == 0
            take = (lower == up) == plt
            arrs = [jnp.where(take, pa, a) for pa, a in zip(p, arrs)]
    sc = arrs[0][:16]
    x0 = arrs[2][:16]
    y0 = arrs[3][:16]
    x1 = arrs[4][:16]
    y1 = arrs[5][:16]

    area = (x1 - x0) * (y1 - y0)
    best = jnp.full((16, L), -1.0, f32)
    mg0 = jnp.zeros((16, L), f32)
    mg1 = jnp.zeros((16, L), f32)
    mg2 = jnp.ones((16, L), f32)
    mg3 = jnp.ones((16, L), f32)
    for g in range(_G):
        g0 = gt_ref[g, 0]
        g1 = gt_ref[g, 1]
        g2 = gt_ref[g, 2]
        g3 = gt_ref[g, 3]
        ga = (g2 - g0) * (g3 - g1)
        iw = jnp.maximum(jnp.minimum(x1, g2) - jnp.maximum(x0, g0), 0.0)
        ih = jnp.maximum(jnp.minimum(y1, g3) - jnp.maximum(y0, g1), 0.0)
        inter = iw * ih
        iou = inter / (area + ga - inter)
        upd = iou > best
        best = jnp.where(upd, iou, best)
        mg0 = jnp.where(upd, g0, mg0)
        mg1 = jnp.where(upd, g1, mg1)
        mg2 = jnp.where(upd, g2, mg2)
        mg3 = jnp.where(upd, g3, mg3)
    pw = x1 - x0
    ph = y1 - y0
    px = x0 + 0.5 * pw
    py = y0 + 0.5 * ph
    gw = mg2 - mg0
    gh = mg3 - mg1
    gx = mg0 + 0.5 * gw
    gy = mg1 + 0.5 * gh
    tx = (gx - px) / pw
    ty = (gy - py) / ph
    tw = jnp.log(gw / pw)
    th = jnp.log(gh / ph)

    c_ref[...] = jnp.zeros((2048, 2048), bf16)
    li_t = lax.broadcasted_iota(i32, (L, L), 0)
    lj_t = lax.broadcasted_iota(i32, (L, L), 1)
    sarea = area * (_T / (1.0 + _T))
    for si in range(16):
        rx0 = x0[si:si + 1, :].T
        ry0 = y0[si:si + 1, :].T
        rx1 = x1[si:si + 1, :].T
        ry1 = y1[si:si + 1, :].T
        ra = sarea[si:si + 1, :].T
        for sj in range(si, 16):
            iw = jnp.maximum(jnp.minimum(rx1, x1[sj:sj + 1, :])
                             - jnp.maximum(rx0, x0[sj:sj + 1, :]), 0.0)
            ih = jnp.maximum(jnp.minimum(ry1, y1[sj:sj + 1, :])
                             - jnp.maximum(ry0, y0[sj:sj + 1, :]), 0.0)
            conf = iw * ih > ra + sarea[sj:sj + 1, :]
            if si == sj:
                conf = conf & (lj_t > li_t)
            c_ref[si * L:(si + 1) * L, sj * L:(sj + 1) * L] = conf.astype(bf16)

    def cond(c):
        alive, kept = c
        return jnp.sum(alive) > 0.0

    def rbody(c):
        alive, kept = c
        al8 = jnp.broadcast_to(alive, (8, 2048)).astype(bf16)
        ear = jnp.dot(al8, c_ref[...], preferred_element_type=f32)[0:1]
        frontier = (alive > 0.0) & (ear <= 0.5)
        fr8 = jnp.broadcast_to(frontier.astype(f32), (8, 2048)).astype(bf16)
        spread = jnp.dot(fr8, c_ref[...], preferred_element_type=f32)[0:1] > 0.5
        kept = jnp.where(frontier, 1.0, kept)
        alive = jnp.where(frontier | spread, 0.0, alive)
        return alive, kept

    alive0 = jnp.ones((1, 2048), f32)
    kept0 = jnp.zeros((1, 2048), f32)
    _, kept = lax.while_loop(cond, rbody, (alive0, kept0))

    l2 = lax.broadcasted_iota(i32, (1, 2048), 1)
    keptm = (kept > 0.0) & (l2 < _PRE)
    kv = keptm.astype(f32)
    run = kv
    for dd in (1, 2, 4, 8, 16, 32, 64, 128, 256, 512, 1024):
        run = run + jnp.where(l2 >= dd, pltpu.roll(run, dd, 1), 0.0)
    pos = run - kv

    o_io = lax.broadcasted_iota(i32, (L, 1024), 1).astype(f32)
    for si in range(16):
        ps = pos[0:1, si * L:(si + 1) * L].T
        ks = kv[0:1, si * L:(si + 1) * L].T
        pt_ref[si * L:(si + 1) * L, :] = \
            ((ps == o_io) & (ks > 0.5) & (ps < float(_POST))).astype(bf16)

    r16 = lax.broadcasted_iota(i32, (16, L), 0) * L \
        + lax.broadcasted_iota(i32, (16, L), 1)
    sc_s = jnp.where(r16 < _PRE, sc, 0.0)

    acc = jnp.zeros((16, 1024), f32)
    for si in range(16):
        dts = jnp.concatenate(
            [x0[si:si + 1], y0[si:si + 1], x1[si:si + 1], y1[si:si + 1],
             tx[si:si + 1], ty[si:si + 1], tw[si:si + 1], th[si:si + 1],
             sc_s[si:si + 1], jnp.zeros((7, L), f32)], axis=0)
        acc = acc + jnp.dot(dts, pt_ref[si * L:(si + 1) * L, :].astype(f32),
                            precision=lax.Precision.HIGHEST,
                            preferred_element_type=f32)
    out_ref[...] = acc.T


def _k3_call(gt_bbox, sc_c, ix_c, x0_c, y0_c, x1_c, y1_c):
    return pl.pallas_call(
        _k3_body,
        out_shape=jax.ShapeDtypeStruct((1024, 16), jnp.float32),
        in_specs=[
            pl.BlockSpec(memory_space=pltpu.SMEM),
            pl.BlockSpec(memory_space=pltpu.VMEM),
            pl.BlockSpec(memory_space=pltpu.VMEM),
            pl.BlockSpec(memory_space=pltpu.VMEM),
            pl.BlockSpec(memory_space=pltpu.VMEM),
            pl.BlockSpec(memory_space=pltpu.VMEM),
            pl.BlockSpec(memory_space=pltpu.VMEM),
        ],
        out_specs=pl.BlockSpec(memory_space=pltpu.VMEM),
        scratch_shapes=[
            pltpu.VMEM((2048, 2048), jnp.bfloat16),
            pltpu.VMEM((2048, 1024), jnp.bfloat16),
        ],
    )(gt_bbox, sc_c.reshape(32, 128), ix_c.reshape(32, 128),
      x0_c.reshape(32, 128), y0_c.reshape(32, 128),
      x1_c.reshape(32, 128), y1_c.reshape(32, 128))


def kernel(boxes, scores, gt_bbox):
    spf = jnp.concatenate(
        [scores, jnp.full((_PADN - _N,), _NEGI, jnp.float32)])
    sp = spf.reshape(_NB, 8, 128)
    tau, needn = _k1_call(sp)
    padb = jnp.broadcast_to(jnp.array([0.0, 0.0, 1.0, 1.0], jnp.float32),
                            (_PADN - _N, 4))
    cols = jnp.concatenate([boxes, padb], axis=0).T
    bx0, by0, bx1, by1 = cols[0], cols[1], cols[2], cols[3]
    sc_c, ix_c, x0_c, y0_c, x1_c, y1_c = _k2_call(
        spf, bx0, by0, bx1, by1, tau, needn)
    out = _k3_call(gt_bbox, sc_c, ix_c, x0_c, y0_c, x1_c, y1_c)
    return out[:_POST, :9]
